# baseline (speedup 1.0000x reference)
import math

import jax
import jax.numpy as jnp
from jax import lax
from jax.experimental import pallas as pl
from jax.experimental.pallas import tpu as pltpu
from jax.experimental.pallas import tpu_sc as plsc

N = 10000
D = 128
H = 8
E = 320000
EN = E + N
NPAD = 10112
EPAD = 330240
EPT = EPAD // 32
NPT = NPAD // 16
CA = 40
NA = EPT // CA
CB = 24
NB = EPT // CB
PK = H * D // 2
DUM = N
_BN = 1.0 / math.sqrt(1.0 + 1e-5)

f32 = jnp.float32
i32 = jnp.int32
bf16 = jnp.bfloat16


def _elu(x):
    return jnp.where(x > 0, x, jnp.exp(jnp.minimum(x, 0.0)) - 1.0)


def _pre_body(ctrl, mask, ipw, ipb, lng, lnb, rw, rb, x0_o, res_o):
    t = ctrl[:, :] * ipw[:, :] + ipb[:, :]
    m = jnp.mean(t, axis=-1, keepdims=True)
    v = jnp.mean((t - m) ** 2, axis=-1, keepdims=True)
    t = (t - m) / jnp.sqrt(v + 1e-5) * lng[:, :] + lnb[:, :]
    x0 = _elu(t) * mask[:, :]
    x0_o[:, :] = x0
    res_o[:, :] = jnp.dot(x0, rw[:, :], preferred_element_type=f32) + rb[:, :]


def _tc_pre(ctrl, mask, ipw, ipb, lng, lnb, rw, rb):
    g = NPAD // NPT
    return pl.pallas_call(
        _pre_body,
        grid=(g,),
        in_specs=[
            pl.BlockSpec((NPT, 1), lambda i: (i, 0)),
            pl.BlockSpec((NPT, 1), lambda i: (i, 0)),
            pl.BlockSpec((1, D), lambda i: (0, 0)),
            pl.BlockSpec((1, D), lambda i: (0, 0)),
            pl.BlockSpec((1, D), lambda i: (0, 0)),
            pl.BlockSpec((1, D), lambda i: (0, 0)),
            pl.BlockSpec((D, D), lambda i: (0, 0)),
            pl.BlockSpec((1, D), lambda i: (0, 0)),
        ],
        out_specs=[
            pl.BlockSpec((NPT, D), lambda i: (i, 0)),
            pl.BlockSpec((NPT, D), lambda i: (i, 0)),
        ],
        out_shape=[
            jax.ShapeDtypeStruct((NPAD, D), f32),
            jax.ShapeDtypeStruct((NPAD, D), f32),
        ],
    )(ctrl, mask, ipw, ipb, lng, lnb, rw, rb)


def _lpre_body(x, w, asf, adf, xpb_o, als_o, ald_o):
    xp = jnp.dot(x[:, :], w[:, :], preferred_element_type=f32)
    xpb_o[:, :] = xp.astype(bf16)
    r = lax.broadcasted_iota(i32, (H * D, D), 0) // D
    c = lax.broadcasted_iota(i32, (H * D, D), 1)
    sel = (r == c).astype(f32)
    als_o[:, :] = jnp.dot(xp * asf[:, :], sel, preferred_element_type=f32)
    ald_o[:, :] = jnp.dot(xp * adf[:, :], sel, preferred_element_type=f32)


def _tc_lpre(x, w, asf, adf):
    g = NPAD // NPT
    return pl.pallas_call(
        _lpre_body,
        grid=(g,),
        in_specs=[
            pl.BlockSpec((NPT, D), lambda i: (i, 0)),
            pl.BlockSpec((D, H * D), lambda i: (0, 0)),
            pl.BlockSpec((1, H * D), lambda i: (0, 0)),
            pl.BlockSpec((1, H * D), lambda i: (0, 0)),
        ],
        out_specs=[
            pl.BlockSpec((NPT, H * D), lambda i: (i, 0)),
            pl.BlockSpec((NPT, D), lambda i: (i, 0)),
            pl.BlockSpec((NPT, D), lambda i: (i, 0)),
        ],
        out_shape=[
            jax.ShapeDtypeStruct((NPAD, H * D), bf16),
            jax.ShapeDtypeStruct((NPAD, D), f32),
            jax.ShapeDtypeStruct((NPAD, D), f32),
        ],
    )(x, w, asf, adf)


def _rden_body(d0, d1, ald, o):
    logr = -jnp.log(d0[:, :] + d1[:, :] + 1e-16)
    col = lax.broadcasted_iota(i32, (NPT, D), 1)
    logr = jnp.where(col < 16, logr, 0.0)
    r = lax.broadcasted_iota(i32, (D, D), 0)
    c = lax.broadcasted_iota(i32, (D, D), 1)
    shift = (c == r + 16).astype(f32)
    o[:, :] = ald[:, :] + jnp.dot(logr, shift, preferred_element_type=f32)


def _tc_rden(d0, d1, ald):
    g = NPAD // NPT
    return pl.pallas_call(
        _rden_body,
        grid=(g,),
        in_specs=[
            pl.BlockSpec((NPT, D), lambda i: (i, 0)),
            pl.BlockSpec((NPT, D), lambda i: (i, 0)),
            pl.BlockSpec((NPT, D), lambda i: (i, 0)),
        ],
        out_specs=pl.BlockSpec((NPT, D), lambda i: (i, 0)),
        out_shape=jax.ShapeDtypeStruct((NPAD, D), f32),
    )(d0, d1, ald)


def _comb_body(p0, p1, b, g_, b_, o):
    agg = (p0[:, :] + p1[:, :]) * (1.0 / H) + b[:, :]
    o[:, :] = _elu(agg * (_BN * g_[:, :]) + b_[:, :])


def _tc_combine(p0, p1, b, bng, bnb):
    g = NPAD // NPT
    return pl.pallas_call(
        _comb_body,
        grid=(g,),
        in_specs=[
            pl.BlockSpec((NPT, D), lambda i: (i, 0)),
            pl.BlockSpec((NPT, D), lambda i: (i, 0)),
            pl.BlockSpec((1, D), lambda i: (0, 0)),
            pl.BlockSpec((1, D), lambda i: (0, 0)),
            pl.BlockSpec((1, D), lambda i: (0, 0)),
        ],
        out_specs=pl.BlockSpec((NPT, D), lambda i: (i, 0)),
        out_shape=jax.ShapeDtypeStruct((NPAD, D), f32),
    )(p0, p1, b, bng, bnb)


def _post_body(p0, p1, res, b, g_, b_, hw, hb, o):
    agg = (p0[:, :] + p1[:, :]) * (1.0 / H) + b[:, :]
    x3 = agg * (_BN * g_[:, :]) + b_[:, :]
    z = _elu(x3 + res[:, :])
    o[:, :] = jnp.dot(z, hw[:, :], preferred_element_type=f32) + hb[:, :]


def _tc_post(p0, p1, res, b, bng, bnb, hw, hb):
    g = NPAD // NPT
    return pl.pallas_call(
        _post_body,
        grid=(g,),
        in_specs=[
            pl.BlockSpec((NPT, D), lambda i: (i, 0)),
            pl.BlockSpec((NPT, D), lambda i: (i, 0)),
            pl.BlockSpec((NPT, D), lambda i: (i, 0)),
            pl.BlockSpec((1, D), lambda i: (0, 0)),
            pl.BlockSpec((1, D), lambda i: (0, 0)),
            pl.BlockSpec((1, D), lambda i: (0, 0)),
            pl.BlockSpec((D, 1), lambda i: (0, 0)),
            pl.BlockSpec((1, 1), lambda i: (0, 0)),
        ],
        out_specs=pl.BlockSpec((NPT, 1), lambda i: (i, 0)),
        out_shape=jax.ShapeDtypeStruct((NPAD, 1), f32),
    )(p0, p1, res, b, bng, bnb, hw, hb)


_MESH = plsc.VectorSubcoreMesh(core_axis_name="c", subcore_axis_name="s")

_DN = lax.GatherDimensionNumbers(offset_dims=(), collapsed_slice_dims=(0,),
                                 start_index_map=(0,))


def _splat(v, h):
    return lax.gather(v, jnp.full((16, 1), h, i32), _DN, (1,),
                      mode=lax.GatherScatterMode.PROMISE_IN_BOUNDS)


def _bits_to_f32(xi):
    return lax.bitcast_convert_type(xi, f32)


def _sc_a_body(src_hbm, dst_hbm, alst_hbm, aldt_hbm, zden_hbm,
               den_out,
               sv0, sv1, dv0, dv1, sd0, sd1, as0, as1, ad0, ad1,
               eb0, eb1, den_sh, gs0, gs1, ss0, ss1):
    c = lax.axis_index("c")
    s = lax.axis_index("s")
    wid = c * 16 + s
    pltpu.sync_copy(zden_hbm, den_sh.at[pl.ds(s * NPT, NPT)])

    def zr(j, _):
        for k in range(1, D // 16):
            eb0[j, pl.ds(k * 16, 16)] = jnp.zeros((16,), f32)
            eb1[j, pl.ds(k * 16, 16)] = jnp.zeros((16,), f32)
        return 0

    lax.fori_loop(0, CA, zr, 0)
    plsc.subcore_barrier()
    base = wid * EPT

    def issue(off, sv, dv, ag, ad, gs):
        pltpu.sync_copy(src_hbm.at[pl.ds(off, CA)], sv)
        pltpu.sync_copy(dst_hbm.at[pl.ds(off, CA)], dv)
        pltpu.async_copy(alst_hbm.at[sv], ag, gs)
        pltpu.async_copy(aldt_hbm.at[dv], ad, gs)

    issue(base, sv0, dv0, as0, ad0, gs0)
    issue(base + CA, sv1, dv1, as1, ad1, gs1)

    def step(i, half, sv, dv, sd, ag, ad, eb, gs, ss):
        ci = 2 * i + half
        off = base + ci * CA
        pltpu.make_async_copy(alst_hbm.at[sv], ag, gs).wait()
        pltpu.make_async_copy(aldt_hbm.at[dv], ad, gs).wait()

        @pl.when(i > 0)
        def _():
            pltpu.make_async_copy(eb, den_sh.at[sd], ss).wait()

        pltpu.sync_copy(dst_hbm.at[pl.ds(off, CA)], sd)

        @plsc.parallel_loop(0, CA, unroll=8)
        def _(j):
            v = ag[j, pl.ds(0, 16)] + ad[j, pl.ds(0, 16)]
            v = jnp.where(v > 0, v, 0.2 * v)
            eb[j, pl.ds(0, 16)] = jnp.exp(v)
        pltpu.async_copy(eb, den_sh.at[sd], ss, add=True)

        @pl.when(ci + 2 < NA)
        def _():
            issue(off + 2 * CA, sv, dv, ag, ad, gs)

    def it(i, _):
        step(i, 0, sv0, dv0, sd0, as0, ad0, eb0, gs0, ss0)
        step(i, 1, sv1, dv1, sd1, as1, ad1, eb1, gs1, ss1)
        return 0

    lax.fori_loop(0, NA // 2, it, 0)
    pltpu.make_async_copy(eb0, den_sh.at[sd0], ss0).wait()
    pltpu.make_async_copy(eb1, den_sh.at[sd1], ss1).wait()
    plsc.subcore_barrier()
    pltpu.sync_copy(den_sh.at[pl.ds(s * NPT, NPT)],
                    den_out.at[c, pl.ds(s * NPT, NPT)])


def _sc_a(src, dst, alst, aldt, zden):
    return pl.kernel(
        _sc_a_body,
        out_type=jax.ShapeDtypeStruct((2, NPAD, D), f32),
        mesh=_MESH,
        scratch_types=[
            pltpu.VMEM((CA,), i32), pltpu.VMEM((CA,), i32),
            pltpu.VMEM((CA,), i32), pltpu.VMEM((CA,), i32),
            pltpu.VMEM((CA,), i32), pltpu.VMEM((CA,), i32),
            pltpu.VMEM((CA, D), f32), pltpu.VMEM((CA, D), f32),
            pltpu.VMEM((CA, D), f32), pltpu.VMEM((CA, D), f32),
            pltpu.VMEM((CA, D), f32), pltpu.VMEM((CA, D), f32),
            pltpu.VMEM_SHARED((NPAD, D), f32),
            pltpu.SemaphoreType.DMA, pltpu.SemaphoreType.DMA,
            pltpu.SemaphoreType.DMA, pltpu.SemaphoreType.DMA,
        ],
    )(src, dst, alst, aldt, zden)


def _sc_b_body(src_hbm, dst_hbm, xpk_hbm, alst_hbm, t2_hbm, zacc_hbm,
               part_out,
               sv0, sv1, dv0, dv1, sd0, sd1, gb0, gb1, as0, as1,
               ad0, ad1, mg0, mg1,
               acc_sh, gs0, gs1, ss0, ss1):
    c = lax.axis_index("c")
    s = lax.axis_index("s")
    wid = c * 16 + s
    pltpu.sync_copy(zacc_hbm, acc_sh.at[pl.ds(s * NPT, NPT)])
    plsc.subcore_barrier()
    base = wid * EPT

    def issue(off, sv, dv, gb, ag, ad, gs):
        pltpu.sync_copy(src_hbm.at[pl.ds(off, CB)], sv)
        pltpu.sync_copy(dst_hbm.at[pl.ds(off, CB)], dv)
        pltpu.async_copy(xpk_hbm.at[sv], gb, gs)
        pltpu.async_copy(alst_hbm.at[sv], ag, gs)
        pltpu.async_copy(t2_hbm.at[dv], ad, gs)

    issue(base, sv0, dv0, gb0, as0, ad0, gs0)
    issue(base + CB, sv1, dv1, gb1, as1, ad1, gs1)

    def step(i, half, sv, dv, sd, gb, ag, ad, mg, gs, ss):
        ci = 2 * i + half
        off = base + ci * CB
        pltpu.make_async_copy(xpk_hbm.at[sv], gb, gs).wait()
        pltpu.make_async_copy(alst_hbm.at[sv], ag, gs).wait()
        pltpu.make_async_copy(t2_hbm.at[dv], ad, gs).wait()

        @pl.when(i > 0)
        def _():
            pltpu.make_async_copy(mg, acc_sh.at[sd], ss).wait()

        pltpu.sync_copy(dst_hbm.at[pl.ds(off, CB)], sd)

        @plsc.parallel_loop(0, CB, unroll=4)
        def _(j):
            a = ag[j, pl.ds(0, 16)] + ad[j, pl.ds(0, 16)]
            a = jnp.where(a > 0, a, 0.2 * a)
            coef = jnp.exp(a + ad[j, pl.ds(16, 16)])
            chs = [_splat(coef, h) for h in range(H)]
            for w in range(4):
                acc_e = None
                acc_o = None
                for h in range(H):
                    flat = h * 64 + w * 16
                    xi = gb[j, flat // 128, pl.ds(flat % 128, 16)]
                    ev = _bits_to_f32(lax.shift_left(xi, 16))
                    od = _bits_to_f32(jnp.bitwise_and(xi, jnp.int32(-65536)))
                    if h == 0:
                        acc_e = chs[h] * ev
                        acc_o = chs[h] * od
                    else:
                        acc_e = acc_e + chs[h] * ev
                        acc_o = acc_o + chs[h] * od
                mg[j, pl.ds(w * 32, 16)] = acc_e
                mg[j, pl.ds(w * 32 + 16, 16)] = acc_o

        pltpu.async_copy(mg, acc_sh.at[sd], ss, add=True)

        @pl.when(ci + 2 < NB)
        def _():
            issue(off + 2 * CB, sv, dv, gb, ag, ad, gs)

    def it(i, _):
        step(i, 0, sv0, dv0, sd0, gb0, as0, ad0, mg0, gs0, ss0)
        step(i, 1, sv1, dv1, sd1, gb1, as1, ad1, mg1, gs1, ss1)
        return 0

    lax.fori_loop(0, NB // 2, it, 0)
    pltpu.make_async_copy(mg0, acc_sh.at[sd0], ss0).wait()
    pltpu.make_async_copy(mg1, acc_sh.at[sd1], ss1).wait()
    plsc.subcore_barrier()
    pltpu.sync_copy(acc_sh.at[pl.ds(s * NPT, NPT)],
                    part_out.at[c, pl.ds(s * NPT, NPT)])


def _sc_b(src, dst, xpk, alst, t2, zacc):
    return pl.kernel(
        _sc_b_body,
        out_type=jax.ShapeDtypeStruct((2, NPAD, D), f32),
        mesh=_MESH,
        scratch_types=[
            pltpu.VMEM((CB,), i32), pltpu.VMEM((CB,), i32),
            pltpu.VMEM((CB,), i32), pltpu.VMEM((CB,), i32),
            pltpu.VMEM((CB,), i32), pltpu.VMEM((CB,), i32),
            pltpu.VMEM((CB, PK // 128, 128), i32),
            pltpu.VMEM((CB, PK // 128, 128), i32),
            pltpu.VMEM((CB, D), f32), pltpu.VMEM((CB, D), f32),
            pltpu.VMEM((CB, D), f32), pltpu.VMEM((CB, D), f32),
            pltpu.VMEM((CB, D), f32), pltpu.VMEM((CB, D), f32),
            pltpu.VMEM_SHARED((NPAD, D), f32),
            pltpu.SemaphoreType.DMA, pltpu.SemaphoreType.DMA,
            pltpu.SemaphoreType.DMA, pltpu.SemaphoreType.DMA,
        ],
    )(src, dst, xpk, alst, t2, zacc)


def _perm_cols(w):
    return w.reshape(-1, H, 4, 2, 16).transpose(0, 1, 2, 4, 3).reshape(w.shape)


def kernel(ctrl_expr, perturbation_mask, edge_index, ip_W, ip_b, ln_g, ln_b,
           W1, as1, ad1, b1, bn1_g, bn1_b,
           W2, as2, ad2, b2, bn2_g, bn2_b,
           W3, as3, ad3, b3, bn3_g, bn3_b,
           res_W, res_b, head_W, head_b):
    ctrl = jnp.zeros((NPAD, 1), f32).at[:N, 0].set(ctrl_expr)
    mask = jnp.zeros((NPAD, 1), f32).at[:N, 0].set(perturbation_mask)
    loop = jnp.arange(N, dtype=i32)
    padi = jnp.full((EPAD - EN,), DUM, i32)
    src = jnp.concatenate([edge_index[0].astype(i32), loop, padi])
    dst = jnp.concatenate([edge_index[1].astype(i32), loop, padi])
    zacc = jnp.zeros((NPT, D), f32)
    row = lambda v: v.reshape(1, -1)

    x0, resid = _tc_pre(ctrl, mask, ip_W, row(ip_b), row(ln_g), row(ln_b),
                        res_W, row(res_b))

    x = x0
    parts = None
    layers = [(W1, as1, ad1, b1, bn1_g, bn1_b),
              (W2, as2, ad2, b2, bn2_g, bn2_b),
              (W3, as3, ad3, b3, bn3_g, bn3_b)]
    for li, (W, a_s, a_d, b, bg, bb) in enumerate(layers):
        if li > 0:
            _, _, _, b_, bg_, bb_ = layers[li - 1]
            x = _tc_combine(parts[0], parts[1], row(b_), row(bg_), row(bb_))
        Wp = _perm_cols(W)
        asp = _perm_cols(a_s.reshape(1, -1))
        adp = _perm_cols(a_d.reshape(1, -1))
        xpb, alst, aldt = _tc_lpre(x, Wp, asp, adp)
        xpk = lax.bitcast_convert_type(
            xpb.reshape(NPAD, PK, 2), i32).reshape(NPAD, PK // 128, 128)
        den = _sc_a(src, dst, alst, aldt, zacc)
        t2 = _tc_rden(den[0], den[1], aldt)
        parts = _sc_b(src, dst, xpk, alst, t2, zacc)

    out = _tc_post(parts[0], parts[1], resid, row(b3), row(bn3_g), row(bn3_b),
                   head_W, row(head_b))
    return out[:N, 0]


# speedup vs baseline: 1.0350x; 1.0350x over previous
import math

import jax
import jax.numpy as jnp
from jax import lax
from jax.experimental import pallas as pl
from jax.experimental.pallas import tpu as pltpu
from jax.experimental.pallas import tpu_sc as plsc

N = 10000
D = 128
H = 8
E = 320000
EN = E + N
NPAD = 10112
EPAD = 330240
EPT = EPAD // 32
NPT = NPAD // 16
CA = 40
NA = EPT // CA
CB = 24
NB = EPT // CB
PK = H * D // 2
DUM = N
_BN = 1.0 / math.sqrt(1.0 + 1e-5)

f32 = jnp.float32
i32 = jnp.int32
bf16 = jnp.bfloat16


def _elu(x):
    return jnp.where(x > 0, x, jnp.exp(jnp.minimum(x, 0.0)) - 1.0)


def _pre_body(ctrl, mask, ipw, ipb, lng, lnb, rw, rb, x0_o, res_o):
    t = ctrl[:, :] * ipw[:, :] + ipb[:, :]
    m = jnp.mean(t, axis=-1, keepdims=True)
    v = jnp.mean((t - m) ** 2, axis=-1, keepdims=True)
    t = (t - m) / jnp.sqrt(v + 1e-5) * lng[:, :] + lnb[:, :]
    x0 = _elu(t) * mask[:, :]
    x0_o[:, :] = x0
    res_o[:, :] = jnp.dot(x0, rw[:, :], preferred_element_type=f32) + rb[:, :]


def _tc_pre(ctrl, mask, ipw, ipb, lng, lnb, rw, rb):
    g = NPAD // NPT
    return pl.pallas_call(
        _pre_body,
        grid=(g,),
        in_specs=[
            pl.BlockSpec((NPT, 1), lambda i: (i, 0)),
            pl.BlockSpec((NPT, 1), lambda i: (i, 0)),
            pl.BlockSpec((1, D), lambda i: (0, 0)),
            pl.BlockSpec((1, D), lambda i: (0, 0)),
            pl.BlockSpec((1, D), lambda i: (0, 0)),
            pl.BlockSpec((1, D), lambda i: (0, 0)),
            pl.BlockSpec((D, D), lambda i: (0, 0)),
            pl.BlockSpec((1, D), lambda i: (0, 0)),
        ],
        out_specs=[
            pl.BlockSpec((NPT, D), lambda i: (i, 0)),
            pl.BlockSpec((NPT, D), lambda i: (i, 0)),
        ],
        out_shape=[
            jax.ShapeDtypeStruct((NPAD, D), f32),
            jax.ShapeDtypeStruct((NPAD, D), f32),
        ],
    )(ctrl, mask, ipw, ipb, lng, lnb, rw, rb)


def _lpre_body(x, w, asf, adf, xpb_o, als_o, ald_o):
    xp = jnp.dot(x[:, :], w[:, :], preferred_element_type=f32)
    xpb_o[:, :] = xp.astype(bf16)
    r = lax.broadcasted_iota(i32, (H * D, D), 0) // D
    c = lax.broadcasted_iota(i32, (H * D, D), 1)
    sel = (r == c).astype(f32)
    als_o[:, :] = jnp.dot(xp * asf[:, :], sel, preferred_element_type=f32)
    ald_o[:, :] = jnp.dot(xp * adf[:, :], sel, preferred_element_type=f32)


def _tc_lpre(x, w, asf, adf):
    g = NPAD // NPT
    return pl.pallas_call(
        _lpre_body,
        grid=(g,),
        in_specs=[
            pl.BlockSpec((NPT, D), lambda i: (i, 0)),
            pl.BlockSpec((D, H * D), lambda i: (0, 0)),
            pl.BlockSpec((1, H * D), lambda i: (0, 0)),
            pl.BlockSpec((1, H * D), lambda i: (0, 0)),
        ],
        out_specs=[
            pl.BlockSpec((NPT, H * D), lambda i: (i, 0)),
            pl.BlockSpec((NPT, D), lambda i: (i, 0)),
            pl.BlockSpec((NPT, D), lambda i: (i, 0)),
        ],
        out_shape=[
            jax.ShapeDtypeStruct((NPAD, H * D), bf16),
            jax.ShapeDtypeStruct((NPAD, D), f32),
            jax.ShapeDtypeStruct((NPAD, D), f32),
        ],
    )(x, w, asf, adf)


def _rden_body(d0, d1, ald, o):
    logr = -jnp.log(d0[:, :] + d1[:, :] + 1e-16)
    col = lax.broadcasted_iota(i32, (NPT, D), 1)
    logr = jnp.where(col < 16, logr, 0.0)
    r = lax.broadcasted_iota(i32, (D, D), 0)
    c = lax.broadcasted_iota(i32, (D, D), 1)
    shift = (c == r + 16).astype(f32)
    o[:, :] = ald[:, :] + jnp.dot(logr, shift, preferred_element_type=f32)


def _tc_rden(d0, d1, ald):
    g = NPAD // NPT
    return pl.pallas_call(
        _rden_body,
        grid=(g,),
        in_specs=[
            pl.BlockSpec((NPT, D), lambda i: (i, 0)),
            pl.BlockSpec((NPT, D), lambda i: (i, 0)),
            pl.BlockSpec((NPT, D), lambda i: (i, 0)),
        ],
        out_specs=pl.BlockSpec((NPT, D), lambda i: (i, 0)),
        out_shape=jax.ShapeDtypeStruct((NPAD, D), f32),
    )(d0, d1, ald)


def _comb_body(p0, p1, b, g_, b_, o):
    agg = (p0[:, :] + p1[:, :]) * (1.0 / H) + b[:, :]
    o[:, :] = _elu(agg * (_BN * g_[:, :]) + b_[:, :])


def _tc_combine(p0, p1, b, bng, bnb):
    g = NPAD // NPT
    return pl.pallas_call(
        _comb_body,
        grid=(g,),
        in_specs=[
            pl.BlockSpec((NPT, D), lambda i: (i, 0)),
            pl.BlockSpec((NPT, D), lambda i: (i, 0)),
            pl.BlockSpec((1, D), lambda i: (0, 0)),
            pl.BlockSpec((1, D), lambda i: (0, 0)),
            pl.BlockSpec((1, D), lambda i: (0, 0)),
        ],
        out_specs=pl.BlockSpec((NPT, D), lambda i: (i, 0)),
        out_shape=jax.ShapeDtypeStruct((NPAD, D), f32),
    )(p0, p1, b, bng, bnb)


def _post_body(p0, p1, res, b, g_, b_, hw, hb, o):
    agg = (p0[:, :] + p1[:, :]) * (1.0 / H) + b[:, :]
    x3 = agg * (_BN * g_[:, :]) + b_[:, :]
    z = _elu(x3 + res[:, :])
    o[:, :] = jnp.dot(z, hw[:, :], preferred_element_type=f32) + hb[:, :]


def _tc_post(p0, p1, res, b, bng, bnb, hw, hb):
    g = NPAD // NPT
    return pl.pallas_call(
        _post_body,
        grid=(g,),
        in_specs=[
            pl.BlockSpec((NPT, D), lambda i: (i, 0)),
            pl.BlockSpec((NPT, D), lambda i: (i, 0)),
            pl.BlockSpec((NPT, D), lambda i: (i, 0)),
            pl.BlockSpec((1, D), lambda i: (0, 0)),
            pl.BlockSpec((1, D), lambda i: (0, 0)),
            pl.BlockSpec((1, D), lambda i: (0, 0)),
            pl.BlockSpec((D, 1), lambda i: (0, 0)),
            pl.BlockSpec((1, 1), lambda i: (0, 0)),
        ],
        out_specs=pl.BlockSpec((NPT, 1), lambda i: (i, 0)),
        out_shape=jax.ShapeDtypeStruct((NPAD, 1), f32),
    )(p0, p1, res, b, bng, bnb, hw, hb)


_MESH = plsc.VectorSubcoreMesh(core_axis_name="c", subcore_axis_name="s")

_DN = lax.GatherDimensionNumbers(offset_dims=(), collapsed_slice_dims=(0,),
                                 start_index_map=(0,))


def _splat(v, h):
    return lax.gather(v, jnp.full((16, 1), h, i32), _DN, (1,),
                      mode=lax.GatherScatterMode.PROMISE_IN_BOUNDS)


def _bits_to_f32(xi):
    return lax.bitcast_convert_type(xi, f32)


def _sc_a_body(src_hbm, dst_hbm, alst_hbm, aldt_hbm, zden_hbm,
               den_out,
               sv0, sv1, dv0, dv1, sd0, sd1, as0, as1, ad0, ad1,
               eb0, eb1, den_sh, gs0, gs1, ss0, ss1):
    c = lax.axis_index("c")
    s = lax.axis_index("s")
    wid = c * 16 + s
    pltpu.sync_copy(zden_hbm, den_sh.at[pl.ds(s * NPT, NPT)])

    def zr(j, _):
        for k in range(1, D // 16):
            eb0[j, pl.ds(k * 16, 16)] = jnp.zeros((16,), f32)
            eb1[j, pl.ds(k * 16, 16)] = jnp.zeros((16,), f32)
        return 0

    lax.fori_loop(0, CA, zr, 0)
    plsc.subcore_barrier()
    base = wid * EPT

    def issue(off, sv, dv, ag, ad, gs):
        pltpu.sync_copy(src_hbm.at[pl.ds(off, CA)], sv)
        pltpu.sync_copy(dst_hbm.at[pl.ds(off, CA)], dv)
        pltpu.async_copy(alst_hbm.at[sv], ag, gs)
        pltpu.async_copy(aldt_hbm.at[dv], ad, gs)

    issue(base, sv0, dv0, as0, ad0, gs0)
    issue(base + CA, sv1, dv1, as1, ad1, gs1)

    def step(i, half, sv, dv, sd, ag, ad, eb, gs, ss):
        ci = 2 * i + half
        off = base + ci * CA
        pltpu.make_async_copy(alst_hbm.at[sv], ag, gs).wait()
        pltpu.make_async_copy(aldt_hbm.at[dv], ad, gs).wait()

        @pl.when(i > 0)
        def _():
            pltpu.make_async_copy(eb, den_sh.at[sd], ss).wait()

        pltpu.sync_copy(dst_hbm.at[pl.ds(off, CA)], sd)

        @plsc.parallel_loop(0, CA, unroll=4)
        def _(j):
            v = ag[j, pl.ds(0, 16)] + ad[j, pl.ds(0, 16)]
            v = jnp.where(v > 0, v, 0.2 * v)
            eb[j, pl.ds(0, 16)] = jnp.exp(v)
        pltpu.async_copy(eb, den_sh.at[sd], ss, add=True)

        @pl.when(ci + 2 < NA)
        def _():
            issue(off + 2 * CA, sv, dv, ag, ad, gs)

    def it(i, _):
        step(i, 0, sv0, dv0, sd0, as0, ad0, eb0, gs0, ss0)
        step(i, 1, sv1, dv1, sd1, as1, ad1, eb1, gs1, ss1)
        return 0

    lax.fori_loop(0, NA // 2, it, 0)
    pltpu.make_async_copy(eb0, den_sh.at[sd0], ss0).wait()
    pltpu.make_async_copy(eb1, den_sh.at[sd1], ss1).wait()
    plsc.subcore_barrier()
    pltpu.sync_copy(den_sh.at[pl.ds(s * NPT, NPT)],
                    den_out.at[c, pl.ds(s * NPT, NPT)])


def _sc_a(src, dst, alst, aldt, zden):
    return pl.kernel(
        _sc_a_body,
        out_type=jax.ShapeDtypeStruct((2, NPAD, D), f32),
        mesh=_MESH,
        scratch_types=[
            pltpu.VMEM((CA,), i32), pltpu.VMEM((CA,), i32),
            pltpu.VMEM((CA,), i32), pltpu.VMEM((CA,), i32),
            pltpu.VMEM((CA,), i32), pltpu.VMEM((CA,), i32),
            pltpu.VMEM((CA, D), f32), pltpu.VMEM((CA, D), f32),
            pltpu.VMEM((CA, D), f32), pltpu.VMEM((CA, D), f32),
            pltpu.VMEM((CA, D), f32), pltpu.VMEM((CA, D), f32),
            pltpu.VMEM_SHARED((NPAD, D), f32),
            pltpu.SemaphoreType.DMA, pltpu.SemaphoreType.DMA,
            pltpu.SemaphoreType.DMA, pltpu.SemaphoreType.DMA,
        ],
    )(src, dst, alst, aldt, zden)


def _sc_b_body(src_hbm, dst_hbm, xpk_hbm, alst_hbm, t2_hbm, zacc_hbm,
               part_out,
               sv0, sv1, dv0, dv1, sd0, sd1, gb0, gb1, as0, as1,
               ad0, ad1, mg0, mg1,
               acc_sh, gs0, gs1, ss0, ss1):
    c = lax.axis_index("c")
    s = lax.axis_index("s")
    wid = c * 16 + s
    pltpu.sync_copy(zacc_hbm, acc_sh.at[pl.ds(s * NPT, NPT)])
    plsc.subcore_barrier()
    base = wid * EPT

    def issue(off, sv, dv, gb, ag, ad, gs):
        pltpu.sync_copy(src_hbm.at[pl.ds(off, CB)], sv)
        pltpu.sync_copy(dst_hbm.at[pl.ds(off, CB)], dv)
        pltpu.async_copy(xpk_hbm.at[sv], gb, gs)
        pltpu.async_copy(alst_hbm.at[sv], ag, gs)
        pltpu.async_copy(t2_hbm.at[dv], ad, gs)

    issue(base, sv0, dv0, gb0, as0, ad0, gs0)
    issue(base + CB, sv1, dv1, gb1, as1, ad1, gs1)

    def step(i, half, sv, dv, sd, gb, ag, ad, mg, gs, ss):
        ci = 2 * i + half
        off = base + ci * CB
        pltpu.make_async_copy(xpk_hbm.at[sv], gb, gs).wait()
        pltpu.make_async_copy(alst_hbm.at[sv], ag, gs).wait()
        pltpu.make_async_copy(t2_hbm.at[dv], ad, gs).wait()

        @pl.when(i > 0)
        def _():
            pltpu.make_async_copy(mg, acc_sh.at[sd], ss).wait()

        pltpu.sync_copy(dst_hbm.at[pl.ds(off, CB)], sd)

        @plsc.parallel_loop(0, CB, unroll=2)
        def _(j):
            a = ag[j, pl.ds(0, 16)] + ad[j, pl.ds(0, 16)]
            a = jnp.where(a > 0, a, 0.2 * a)
            coef = jnp.exp(a + ad[j, pl.ds(16, 16)])
            chs = [_splat(coef, h) for h in range(H)]
            for w in range(4):
                acc_e = None
                acc_o = None
                for h in range(H):
                    flat = h * 64 + w * 16
                    xi = gb[j, flat // 128, pl.ds(flat % 128, 16)]
                    ev = _bits_to_f32(lax.shift_left(xi, 16))
                    od = _bits_to_f32(jnp.bitwise_and(xi, jnp.int32(-65536)))
                    if h == 0:
                        acc_e = chs[h] * ev
                        acc_o = chs[h] * od
                    else:
                        acc_e = acc_e + chs[h] * ev
                        acc_o = acc_o + chs[h] * od
                mg[j, pl.ds(w * 32, 16)] = acc_e
                mg[j, pl.ds(w * 32 + 16, 16)] = acc_o

        pltpu.async_copy(mg, acc_sh.at[sd], ss, add=True)

        @pl.when(ci + 2 < NB)
        def _():
            issue(off + 2 * CB, sv, dv, gb, ag, ad, gs)

    def it(i, _):
        step(i, 0, sv0, dv0, sd0, gb0, as0, ad0, mg0, gs0, ss0)
        step(i, 1, sv1, dv1, sd1, gb1, as1, ad1, mg1, gs1, ss1)
        return 0

    lax.fori_loop(0, NB // 2, it, 0)
    pltpu.make_async_copy(mg0, acc_sh.at[sd0], ss0).wait()
    pltpu.make_async_copy(mg1, acc_sh.at[sd1], ss1).wait()
    plsc.subcore_barrier()
    pltpu.sync_copy(acc_sh.at[pl.ds(s * NPT, NPT)],
                    part_out.at[c, pl.ds(s * NPT, NPT)])


def _sc_b(src, dst, xpk, alst, t2, zacc):
    return pl.kernel(
        _sc_b_body,
        out_type=jax.ShapeDtypeStruct((2, NPAD, D), f32),
        mesh=_MESH,
        scratch_types=[
            pltpu.VMEM((CB,), i32), pltpu.VMEM((CB,), i32),
            pltpu.VMEM((CB,), i32), pltpu.VMEM((CB,), i32),
            pltpu.VMEM((CB,), i32), pltpu.VMEM((CB,), i32),
            pltpu.VMEM((CB, PK // 128, 128), i32),
            pltpu.VMEM((CB, PK // 128, 128), i32),
            pltpu.VMEM((CB, D), f32), pltpu.VMEM((CB, D), f32),
            pltpu.VMEM((CB, D), f32), pltpu.VMEM((CB, D), f32),
            pltpu.VMEM((CB, D), f32), pltpu.VMEM((CB, D), f32),
            pltpu.VMEM_SHARED((NPAD, D), f32),
            pltpu.SemaphoreType.DMA, pltpu.SemaphoreType.DMA,
            pltpu.SemaphoreType.DMA, pltpu.SemaphoreType.DMA,
        ],
    )(src, dst, xpk, alst, t2, zacc)


def _perm_cols(w):
    return w.reshape(-1, H, 4, 2, 16).transpose(0, 1, 2, 4, 3).reshape(w.shape)


def kernel(ctrl_expr, perturbation_mask, edge_index, ip_W, ip_b, ln_g, ln_b,
           W1, as1, ad1, b1, bn1_g, bn1_b,
           W2, as2, ad2, b2, bn2_g, bn2_b,
           W3, as3, ad3, b3, bn3_g, bn3_b,
           res_W, res_b, head_W, head_b):
    ctrl = jnp.zeros((NPAD, 1), f32).at[:N, 0].set(ctrl_expr)
    mask = jnp.zeros((NPAD, 1), f32).at[:N, 0].set(perturbation_mask)
    loop = jnp.arange(N, dtype=i32)
    padi = jnp.full((EPAD - EN,), DUM, i32)
    src = jnp.concatenate([edge_index[0].astype(i32), loop, padi])
    dst = jnp.concatenate([edge_index[1].astype(i32), loop, padi])
    zacc = jnp.zeros((NPT, D), f32)
    row = lambda v: v.reshape(1, -1)

    x0, resid = _tc_pre(ctrl, mask, ip_W, row(ip_b), row(ln_g), row(ln_b),
                        res_W, row(res_b))

    x = x0
    parts = None
    layers = [(W1, as1, ad1, b1, bn1_g, bn1_b),
              (W2, as2, ad2, b2, bn2_g, bn2_b),
              (W3, as3, ad3, b3, bn3_g, bn3_b)]
    for li, (W, a_s, a_d, b, bg, bb) in enumerate(layers):
        if li > 0:
            _, _, _, b_, bg_, bb_ = layers[li - 1]
            x = _tc_combine(parts[0], parts[1], row(b_), row(bg_), row(bb_))
        Wp = _perm_cols(W)
        asp = _perm_cols(a_s.reshape(1, -1))
        adp = _perm_cols(a_d.reshape(1, -1))
        xpb, alst, aldt = _tc_lpre(x, Wp, asp, adp)
        xpk = lax.bitcast_convert_type(
            xpb.reshape(NPAD, PK, 2), i32).reshape(NPAD, PK // 128, 128)
        den = _sc_a(src, dst, alst, aldt, zacc)
        t2 = _tc_rden(den[0], den[1], aldt)
        parts = _sc_b(src, dst, xpk, alst, t2, zacc)

    out = _tc_post(parts[0], parts[1], resid, row(b3), row(bn3_g), row(bn3_b),
                   head_W, row(head_b))
    return out[:N, 0]


# speedup vs baseline: 1.0390x; 1.0039x over previous
import math

import numpy as np
import jax
import jax.numpy as jnp
from jax import lax
from jax.experimental import pallas as pl
from jax.experimental.pallas import tpu as pltpu
from jax.experimental.pallas import tpu_sc as plsc

N = 10000
D = 128
H = 8
E = 320000
EN = E + N
NPAD = 10112
EPAD = 330240
EPT = EPAD // 32
NPT = NPAD // 16
CA = 40
NA = EPT // CA
CB = 24
NB = EPT // CB
PK = H * D // 2
DUM = N
_BN = 1.0 / math.sqrt(1.0 + 1e-5)

f32 = jnp.float32
i32 = jnp.int32
bf16 = jnp.bfloat16


def _elu(x):
    return jnp.where(x > 0, x, jnp.exp(jnp.minimum(x, 0.0)) - 1.0)


def _pre_body(ctrl, mask, ipw, ipb, lng, lnb, rw, rb, x0_o, res_o):
    t = ctrl[:, :] * ipw[:, :] + ipb[:, :]
    m = jnp.mean(t, axis=-1, keepdims=True)
    v = jnp.mean((t - m) ** 2, axis=-1, keepdims=True)
    t = (t - m) / jnp.sqrt(v + 1e-5) * lng[:, :] + lnb[:, :]
    x0 = _elu(t) * mask[:, :]
    x0_o[:, :] = x0
    res_o[:, :] = jnp.dot(x0, rw[:, :], preferred_element_type=f32) + rb[:, :]


def _tc_pre(ctrl, mask, ipw, ipb, lng, lnb, rw, rb):
    g = NPAD // NPT
    return pl.pallas_call(
        _pre_body,
        grid=(g,),
        in_specs=[
            pl.BlockSpec((NPT, 1), lambda i: (i, 0)),
            pl.BlockSpec((NPT, 1), lambda i: (i, 0)),
            pl.BlockSpec((1, D), lambda i: (0, 0)),
            pl.BlockSpec((1, D), lambda i: (0, 0)),
            pl.BlockSpec((1, D), lambda i: (0, 0)),
            pl.BlockSpec((1, D), lambda i: (0, 0)),
            pl.BlockSpec((D, D), lambda i: (0, 0)),
            pl.BlockSpec((1, D), lambda i: (0, 0)),
        ],
        out_specs=[
            pl.BlockSpec((NPT, D), lambda i: (i, 0)),
            pl.BlockSpec((NPT, D), lambda i: (i, 0)),
        ],
        out_shape=[
            jax.ShapeDtypeStruct((NPAD, D), f32),
            jax.ShapeDtypeStruct((NPAD, D), f32),
        ],
    )(ctrl, mask, ipw, ipb, lng, lnb, rw, rb)


def _lpre_body(x, w, asf, adf, xpb_o, als_o, ald_o):
    xp = jnp.dot(x[:, :], w[:, :], preferred_element_type=f32)
    xpb_o[:, :] = xp.astype(bf16)
    r = lax.broadcasted_iota(i32, (H * D, D), 0) // D
    c = lax.broadcasted_iota(i32, (H * D, D), 1)
    sel = (r == c).astype(f32)
    als_o[:, :] = jnp.dot(xp * asf[:, :], sel, preferred_element_type=f32)
    ald_o[:, :] = jnp.dot(xp * adf[:, :], sel, preferred_element_type=f32)


def _tc_lpre(x, w, asf, adf):
    g = NPAD // NPT
    return pl.pallas_call(
        _lpre_body,
        grid=(g,),
        in_specs=[
            pl.BlockSpec((NPT, D), lambda i: (i, 0)),
            pl.BlockSpec((D, H * D), lambda i: (0, 0)),
            pl.BlockSpec((1, H * D), lambda i: (0, 0)),
            pl.BlockSpec((1, H * D), lambda i: (0, 0)),
        ],
        out_specs=[
            pl.BlockSpec((NPT, H * D), lambda i: (i, 0)),
            pl.BlockSpec((NPT, D), lambda i: (i, 0)),
            pl.BlockSpec((NPT, D), lambda i: (i, 0)),
        ],
        out_shape=[
            jax.ShapeDtypeStruct((NPAD, H * D), bf16),
            jax.ShapeDtypeStruct((NPAD, D), f32),
            jax.ShapeDtypeStruct((NPAD, D), f32),
        ],
    )(x, w, asf, adf)


def _rden_body(d0, d1, o):
    o[:, :] = 1.0 / (d0[:, :] + d1[:, :] + 1e-16)


def _tc_rden(d0, d1):
    g = NPAD // NPT
    return pl.pallas_call(
        _rden_body,
        grid=(g,),
        in_specs=[
            pl.BlockSpec((NPT, D), lambda i: (i, 0)),
            pl.BlockSpec((NPT, D), lambda i: (i, 0)),
        ],
        out_specs=pl.BlockSpec((NPT, D), lambda i: (i, 0)),
        out_shape=jax.ShapeDtypeStruct((NPAD, D), f32),
    )(d0, d1)


def _comb_body(p0, p1, b, g_, b_, o):
    agg = (p0[:, :] + p1[:, :]) * (1.0 / H) + b[:, :]
    o[:, :] = _elu(agg * (_BN * g_[:, :]) + b_[:, :])


def _tc_combine(p0, p1, b, bng, bnb):
    g = NPAD // NPT
    return pl.pallas_call(
        _comb_body,
        grid=(g,),
        in_specs=[
            pl.BlockSpec((NPT, D), lambda i: (i, 0)),
            pl.BlockSpec((NPT, D), lambda i: (i, 0)),
            pl.BlockSpec((1, D), lambda i: (0, 0)),
            pl.BlockSpec((1, D), lambda i: (0, 0)),
            pl.BlockSpec((1, D), lambda i: (0, 0)),
        ],
        out_specs=pl.BlockSpec((NPT, D), lambda i: (i, 0)),
        out_shape=jax.ShapeDtypeStruct((NPAD, D), f32),
    )(p0, p1, b, bng, bnb)


def _post_body(p0, p1, res, b, g_, b_, hw, hb, o):
    agg = (p0[:, :] + p1[:, :]) * (1.0 / H) + b[:, :]
    x3 = agg * (_BN * g_[:, :]) + b_[:, :]
    z = _elu(x3 + res[:, :])
    o[:, :] = jnp.dot(z, hw[:, :], preferred_element_type=f32) + hb[:, :]


def _tc_post(p0, p1, res, b, bng, bnb, hw, hb):
    g = NPAD // NPT
    return pl.pallas_call(
        _post_body,
        grid=(g,),
        in_specs=[
            pl.BlockSpec((NPT, D), lambda i: (i, 0)),
            pl.BlockSpec((NPT, D), lambda i: (i, 0)),
            pl.BlockSpec((NPT, D), lambda i: (i, 0)),
            pl.BlockSpec((1, D), lambda i: (0, 0)),
            pl.BlockSpec((1, D), lambda i: (0, 0)),
            pl.BlockSpec((1, D), lambda i: (0, 0)),
            pl.BlockSpec((D, 1), lambda i: (0, 0)),
            pl.BlockSpec((1, 1), lambda i: (0, 0)),
        ],
        out_specs=pl.BlockSpec((NPT, 1), lambda i: (i, 0)),
        out_shape=jax.ShapeDtypeStruct((NPAD, 1), f32),
    )(p0, p1, res, b, bng, bnb, hw, hb)


_MESH = plsc.VectorSubcoreMesh(core_axis_name="c", subcore_axis_name="s")

_DN = lax.GatherDimensionNumbers(offset_dims=(), collapsed_slice_dims=(0,),
                                 start_index_map=(0,))


def _splat(v, h):
    return lax.gather(v, jnp.full((16, 1), h, i32), _DN, (1,),
                      mode=lax.GatherScatterMode.PROMISE_IN_BOUNDS)


def _bits_to_f32(xi):
    return lax.bitcast_convert_type(xi, f32)


def _sc_a_body(src_hbm, dst_hbm, alst_hbm, aldt_hbm, zden_hbm,
               den_out, e1d_out,
               sv0, sv1, dv0, dv1, sd0, sd1, as0, as1, ad0, ad1,
               eb0, eb1, ev0, ev1, den_sh,
               gs0, gs1, ss0, ss1, es0, es1):
    c = lax.axis_index("c")
    s = lax.axis_index("s")
    wid = c * 16 + s
    pltpu.sync_copy(zden_hbm, den_sh.at[pl.ds(s * NPT, NPT)])

    def zr(j, _):
        for k in range(1, D // 16):
            eb0[j, pl.ds(k * 16, 16)] = jnp.zeros((16,), f32)
            eb1[j, pl.ds(k * 16, 16)] = jnp.zeros((16,), f32)
        return 0

    lax.fori_loop(0, CA, zr, 0)
    plsc.subcore_barrier()
    base = wid * EPT

    def issue(off, sv, dv, ag, ad, gs):
        pltpu.sync_copy(src_hbm.at[pl.ds(off, CA)], sv)
        pltpu.sync_copy(dst_hbm.at[pl.ds(off, CA)], dv)
        pltpu.async_copy(alst_hbm.at[sv], ag, gs)
        pltpu.async_copy(aldt_hbm.at[dv], ad, gs)

    issue(base, sv0, dv0, as0, ad0, gs0)
    issue(base + CA, sv1, dv1, as1, ad1, gs1)

    def step(i, half, sv, dv, sd, ag, ad, eb, ev, gs, ss, es):
        ci = 2 * i + half
        off = base + ci * CA
        pltpu.make_async_copy(alst_hbm.at[sv], ag, gs).wait()
        pltpu.make_async_copy(aldt_hbm.at[dv], ad, gs).wait()

        @pl.when(i > 0)
        def _():
            pltpu.make_async_copy(eb, den_sh.at[sd], ss).wait()
            pltpu.make_async_copy(
                ev, e1d_out.at[pl.ds(0, CA * 8)], es).wait()

        pltpu.sync_copy(dst_hbm.at[pl.ds(off, CA)], sd)

        @plsc.parallel_loop(0, CA, unroll=4)
        def _(j):
            v = ag[j, pl.ds(0, 16)] + ad[j, pl.ds(0, 16)]
            v = jnp.where(v > 0, v, 0.2 * v)
            eb[j, pl.ds(0, 16)] = jnp.exp(v)

        @plsc.parallel_loop(0, CA // 2, unroll=2)
        def _(p):
            lane = lax.iota(i32, 16)
            a = eb[2 * p, pl.ds(0, 16)]
            bvec = eb[2 * p + 1, pl.ds(0, 16)]
            sh = lax.gather(bvec, jnp.maximum(lane - 8, 0).reshape(16, 1),
                            _DN, (1,),
                            mode=lax.GatherScatterMode.PROMISE_IN_BOUNDS)
            ev[pl.ds(p * 16, 16)] = jnp.where(lane < 8, a, sh)

        pltpu.async_copy(eb, den_sh.at[sd], ss, add=True)
        pltpu.async_copy(ev, e1d_out.at[pl.ds(off * 8, CA * 8)], es)

        @pl.when(ci + 2 < NA)
        def _():
            issue(off + 2 * CA, sv, dv, ag, ad, gs)

    def it(i, _):
        step(i, 0, sv0, dv0, sd0, as0, ad0, eb0, ev0, gs0, ss0, es0)
        step(i, 1, sv1, dv1, sd1, as1, ad1, eb1, ev1, gs1, ss1, es1)
        return 0

    lax.fori_loop(0, NA // 2, it, 0)
    pltpu.make_async_copy(eb0, den_sh.at[sd0], ss0).wait()
    pltpu.make_async_copy(eb1, den_sh.at[sd1], ss1).wait()
    pltpu.make_async_copy(ev0, e1d_out.at[pl.ds(0, CA * 8)], es0).wait()
    pltpu.make_async_copy(ev1, e1d_out.at[pl.ds(0, CA * 8)], es1).wait()
    plsc.subcore_barrier()
    pltpu.sync_copy(den_sh.at[pl.ds(s * NPT, NPT)],
                    den_out.at[c, pl.ds(s * NPT, NPT)])


def _sc_a(src, dst, alst, aldt, zden):
    return pl.kernel(
        _sc_a_body,
        out_type=(
            jax.ShapeDtypeStruct((2, NPAD, D), f32),
            jax.ShapeDtypeStruct((EPAD * 8,), f32),
        ),
        mesh=_MESH,
        scratch_types=[
            pltpu.VMEM((CA,), i32), pltpu.VMEM((CA,), i32),
            pltpu.VMEM((CA,), i32), pltpu.VMEM((CA,), i32),
            pltpu.VMEM((CA,), i32), pltpu.VMEM((CA,), i32),
            pltpu.VMEM((CA, D), f32), pltpu.VMEM((CA, D), f32),
            pltpu.VMEM((CA, D), f32), pltpu.VMEM((CA, D), f32),
            pltpu.VMEM((CA, D), f32), pltpu.VMEM((CA, D), f32),
            pltpu.VMEM((CA * 8,), f32), pltpu.VMEM((CA * 8,), f32),
            pltpu.VMEM_SHARED((NPAD, D), f32),
            pltpu.SemaphoreType.DMA, pltpu.SemaphoreType.DMA,
            pltpu.SemaphoreType.DMA, pltpu.SemaphoreType.DMA,
            pltpu.SemaphoreType.DMA, pltpu.SemaphoreType.DMA,
        ],
    )(src, dst, alst, aldt, zden)


def _sc_b_body(src_hbm, dst_hbm, xpk_hbm, rden_hbm, e1d_hbm, zacc_hbm,
               part_out,
               sv0, sv1, dv0, dv1, sd0, sd1, gb0, gb1, ev0, ev1,
               ad0, ad1, mg0, mg1,
               acc_sh, gs0, gs1, ss0, ss1):
    c = lax.axis_index("c")
    s = lax.axis_index("s")
    wid = c * 16 + s
    pltpu.sync_copy(zacc_hbm, acc_sh.at[pl.ds(s * NPT, NPT)])
    plsc.subcore_barrier()
    base = wid * EPT

    def issue(off, sv, dv, gb, evb, ad, gs):
        pltpu.sync_copy(src_hbm.at[pl.ds(off, CB)], sv)
        pltpu.sync_copy(dst_hbm.at[pl.ds(off, CB)], dv)
        pltpu.async_copy(xpk_hbm.at[sv], gb, gs)
        pltpu.async_copy(e1d_hbm.at[pl.ds(off * 8, CB * 8)],
                         evb.at[pl.ds(0, CB * 8)], gs)
        pltpu.async_copy(rden_hbm.at[dv], ad, gs)

    issue(base, sv0, dv0, gb0, ev0, ad0, gs0)
    issue(base + CB, sv1, dv1, gb1, ev1, ad1, gs1)

    def step(i, half, sv, dv, sd, gb, evb, ad, mg, gs, ss):
        ci = 2 * i + half
        off = base + ci * CB
        pltpu.make_async_copy(xpk_hbm.at[sv], gb, gs).wait()
        pltpu.make_async_copy(e1d_hbm.at[pl.ds(off * 8, CB * 8)],
                              evb.at[pl.ds(0, CB * 8)], gs).wait()
        pltpu.make_async_copy(rden_hbm.at[dv], ad, gs).wait()

        @pl.when(i > 0)
        def _():
            pltpu.make_async_copy(mg, acc_sh.at[sd], ss).wait()

        pltpu.sync_copy(dst_hbm.at[pl.ds(off, CB)], sd)

        @plsc.parallel_loop(0, CB, unroll=2)
        def _(j):
            coef = evb[pl.ds(j * 8, 16)] * ad[j, pl.ds(0, 16)]
            chs = [_splat(coef, h) for h in range(H)]
            for w in range(4):
                acc_e = None
                acc_o = None
                for h in range(H):
                    flat = h * 64 + w * 16
                    xi = gb[j, flat // 128, pl.ds(flat % 128, 16)]
                    ev = _bits_to_f32(lax.shift_left(xi, 16))
                    od = _bits_to_f32(jnp.bitwise_and(xi, jnp.int32(-65536)))
                    if h == 0:
                        acc_e = chs[h] * ev
                        acc_o = chs[h] * od
                    else:
                        acc_e = acc_e + chs[h] * ev
                        acc_o = acc_o + chs[h] * od
                mg[j, pl.ds(w * 32, 16)] = acc_e
                mg[j, pl.ds(w * 32 + 16, 16)] = acc_o

        pltpu.async_copy(mg, acc_sh.at[sd], ss, add=True)

        @pl.when(ci + 2 < NB)
        def _():
            issue(off + 2 * CB, sv, dv, gb, evb, ad, gs)

    def it(i, _):
        step(i, 0, sv0, dv0, sd0, gb0, ev0, ad0, mg0, gs0, ss0)
        step(i, 1, sv1, dv1, sd1, gb1, ev1, ad1, mg1, gs1, ss1)
        return 0

    lax.fori_loop(0, NB // 2, it, 0)
    pltpu.make_async_copy(mg0, acc_sh.at[sd0], ss0).wait()
    pltpu.make_async_copy(mg1, acc_sh.at[sd1], ss1).wait()
    plsc.subcore_barrier()
    pltpu.sync_copy(acc_sh.at[pl.ds(s * NPT, NPT)],
                    part_out.at[c, pl.ds(s * NPT, NPT)])


def _sc_b(src, dst, xpk, rden, e1d, zacc):
    return pl.kernel(
        _sc_b_body,
        out_type=jax.ShapeDtypeStruct((2, NPAD, D), f32),
        mesh=_MESH,
        scratch_types=[
            pltpu.VMEM((CB,), i32), pltpu.VMEM((CB,), i32),
            pltpu.VMEM((CB,), i32), pltpu.VMEM((CB,), i32),
            pltpu.VMEM((CB,), i32), pltpu.VMEM((CB,), i32),
            pltpu.VMEM((CB, PK // 128, 128), i32),
            pltpu.VMEM((CB, PK // 128, 128), i32),
            pltpu.VMEM((CB * 8 + 16,), f32), pltpu.VMEM((CB * 8 + 16,), f32),
            pltpu.VMEM((CB, D), f32), pltpu.VMEM((CB, D), f32),
            pltpu.VMEM((CB, D), f32), pltpu.VMEM((CB, D), f32),
            pltpu.VMEM_SHARED((NPAD, D), f32),
            pltpu.SemaphoreType.DMA, pltpu.SemaphoreType.DMA,
            pltpu.SemaphoreType.DMA, pltpu.SemaphoreType.DMA,
        ],
    )(src, dst, xpk, rden, e1d, zacc)


def _perm_cols(w):
    return w.reshape(-1, H, 4, 2, 16).transpose(0, 1, 2, 4, 3).reshape(w.shape)


def kernel(ctrl_expr, perturbation_mask, edge_index, ip_W, ip_b, ln_g, ln_b,
           W1, as1, ad1, b1, bn1_g, bn1_b,
           W2, as2, ad2, b2, bn2_g, bn2_b,
           W3, as3, ad3, b3, bn3_g, bn3_b,
           res_W, res_b, head_W, head_b):
    ctrl = jnp.zeros((NPAD, 1), f32).at[:N, 0].set(ctrl_expr)
    mask = jnp.zeros((NPAD, 1), f32).at[:N, 0].set(perturbation_mask)
    loop = jnp.arange(N, dtype=i32)
    padi = jnp.full((EPAD - EN,), DUM, i32)
    src = jnp.concatenate([edge_index[0].astype(i32), loop, padi])
    dst = jnp.concatenate([edge_index[1].astype(i32), loop, padi])
    zacc = jnp.zeros((NPT, D), f32)
    row = lambda v: v.reshape(1, -1)

    x0, resid = _tc_pre(ctrl, mask, ip_W, row(ip_b), row(ln_g), row(ln_b),
                        res_W, row(res_b))

    x = x0
    parts = None
    layers = [(W1, as1, ad1, b1, bn1_g, bn1_b),
              (W2, as2, ad2, b2, bn2_g, bn2_b),
              (W3, as3, ad3, b3, bn3_g, bn3_b)]
    for li, (W, a_s, a_d, b, bg, bb) in enumerate(layers):
        if li > 0:
            _, _, _, b_, bg_, bb_ = layers[li - 1]
            x = _tc_combine(parts[0], parts[1], row(b_), row(bg_), row(bb_))
        Wp = _perm_cols(W)
        asp = _perm_cols(a_s.reshape(1, -1))
        adp = _perm_cols(a_d.reshape(1, -1))
        xpb, alst, aldt = _tc_lpre(x, Wp, asp, adp)
        xpk = lax.bitcast_convert_type(
            xpb.reshape(NPAD, PK, 2), i32).reshape(NPAD, PK // 128, 128)
        den, e1d = _sc_a(src, dst, alst, aldt, zacc)
        rden = _tc_rden(den[0], den[1])
        parts = _sc_b(src, dst, xpk, rden, e1d, zacc)

    out = _tc_post(parts[0], parts[1], resid, row(b3), row(bn3_g), row(bn3_b),
                   head_W, row(head_b))
    return out[:N, 0]


# speedup vs baseline: 1.2454x; 1.1986x over previous
import math

import numpy as np
import jax
import jax.numpy as jnp
from jax import lax
from jax.experimental import pallas as pl
from jax.experimental.pallas import tpu as pltpu
from jax.experimental.pallas import tpu_sc as plsc

N = 10000
D = 128
H = 8
E = 320000
EN = E + N
NPAD = 10112
EPAD = 330240
EPT = EPAD // 32
NPT = NPAD // 16
CA = 40
NA = EPT // CA
CB = 24
NB = EPT // CB
PK = H * D // 2
DUM = N
_BN = 1.0 / math.sqrt(1.0 + 1e-5)

f32 = jnp.float32
i32 = jnp.int32
bf16 = jnp.bfloat16


def _elu(x):
    return jnp.where(x > 0, x, jnp.exp(jnp.minimum(x, 0.0)) - 1.0)


def _pre_body(ctrl, mask, ipw, ipb, lng, lnb, rw, rb, x0_o, res_o):
    t = ctrl[:, :] * ipw[:, :] + ipb[:, :]
    m = jnp.mean(t, axis=-1, keepdims=True)
    v = jnp.mean((t - m) ** 2, axis=-1, keepdims=True)
    t = (t - m) / jnp.sqrt(v + 1e-5) * lng[:, :] + lnb[:, :]
    x0 = _elu(t) * mask[:, :]
    x0_o[:, :] = x0
    res_o[:, :] = jnp.dot(x0, rw[:, :], preferred_element_type=f32) + rb[:, :]


def _tc_pre(ctrl, mask, ipw, ipb, lng, lnb, rw, rb):
    g = NPAD // NPT
    return pl.pallas_call(
        _pre_body,
        grid=(g,),
        in_specs=[
            pl.BlockSpec((NPT, 1), lambda i: (i, 0)),
            pl.BlockSpec((NPT, 1), lambda i: (i, 0)),
            pl.BlockSpec((1, D), lambda i: (0, 0)),
            pl.BlockSpec((1, D), lambda i: (0, 0)),
            pl.BlockSpec((1, D), lambda i: (0, 0)),
            pl.BlockSpec((1, D), lambda i: (0, 0)),
            pl.BlockSpec((D, D), lambda i: (0, 0)),
            pl.BlockSpec((1, D), lambda i: (0, 0)),
        ],
        out_specs=[
            pl.BlockSpec((NPT, D), lambda i: (i, 0)),
            pl.BlockSpec((NPT, D), lambda i: (i, 0)),
        ],
        out_shape=[
            jax.ShapeDtypeStruct((NPAD, D), f32),
            jax.ShapeDtypeStruct((NPAD, D), f32),
        ],
    )(ctrl, mask, ipw, ipb, lng, lnb, rw, rb)


def _lpre_body(x, w, asf, adf, xpb_o, als_o, ald_o):
    xp = jnp.dot(x[:, :], w[:, :], preferred_element_type=f32)
    xpb_o[:, :] = xp.astype(bf16)
    r = lax.broadcasted_iota(i32, (H * D, D), 0) // D
    c = lax.broadcasted_iota(i32, (H * D, D), 1)
    sel = (r == c).astype(f32)
    als_o[:, :] = jnp.dot(xp * asf[:, :], sel, preferred_element_type=f32)
    ald_o[:, :] = jnp.dot(xp * adf[:, :], sel, preferred_element_type=f32)


def _tc_lpre(x, w, asf, adf):
    g = NPAD // NPT
    return pl.pallas_call(
        _lpre_body,
        grid=(g,),
        in_specs=[
            pl.BlockSpec((NPT, D), lambda i: (i, 0)),
            pl.BlockSpec((D, H * D), lambda i: (0, 0)),
            pl.BlockSpec((1, H * D), lambda i: (0, 0)),
            pl.BlockSpec((1, H * D), lambda i: (0, 0)),
        ],
        out_specs=[
            pl.BlockSpec((NPT, H * D), lambda i: (i, 0)),
            pl.BlockSpec((NPT, D), lambda i: (i, 0)),
            pl.BlockSpec((NPT, D), lambda i: (i, 0)),
        ],
        out_shape=[
            jax.ShapeDtypeStruct((NPAD, H * D), bf16),
            jax.ShapeDtypeStruct((NPAD, D), f32),
            jax.ShapeDtypeStruct((NPAD, D), f32),
        ],
    )(x, w, asf, adf)


def _rden_body(d0, d1, o):
    o[:, :] = 1.0 / (d0[:, :] + d1[:, :] + 1e-16)


def _tc_rden(d0, d1):
    g = NPAD // NPT
    return pl.pallas_call(
        _rden_body,
        grid=(g,),
        in_specs=[
            pl.BlockSpec((NPT, D), lambda i: (i, 0)),
            pl.BlockSpec((NPT, D), lambda i: (i, 0)),
        ],
        out_specs=pl.BlockSpec((NPT, D), lambda i: (i, 0)),
        out_shape=jax.ShapeDtypeStruct((NPAD, D), f32),
    )(d0, d1)


def _comb_body(p0, p1, b, g_, b_, o):
    agg = (p0[:, :] + p1[:, :]) * (1.0 / H) + b[:, :]
    o[:, :] = _elu(agg * (_BN * g_[:, :]) + b_[:, :])


def _tc_combine(p0, p1, b, bng, bnb):
    g = NPAD // NPT
    return pl.pallas_call(
        _comb_body,
        grid=(g,),
        in_specs=[
            pl.BlockSpec((NPT, D), lambda i: (i, 0)),
            pl.BlockSpec((NPT, D), lambda i: (i, 0)),
            pl.BlockSpec((1, D), lambda i: (0, 0)),
            pl.BlockSpec((1, D), lambda i: (0, 0)),
            pl.BlockSpec((1, D), lambda i: (0, 0)),
        ],
        out_specs=pl.BlockSpec((NPT, D), lambda i: (i, 0)),
        out_shape=jax.ShapeDtypeStruct((NPAD, D), f32),
    )(p0, p1, b, bng, bnb)


def _post_body(p0, p1, res, b, g_, b_, hw, hb, o):
    agg = (p0[:, :] + p1[:, :]) * (1.0 / H) + b[:, :]
    x3 = agg * (_BN * g_[:, :]) + b_[:, :]
    z = _elu(x3 + res[:, :])
    o[:, :] = jnp.dot(z, hw[:, :], preferred_element_type=f32) + hb[:, :]


def _tc_post(p0, p1, res, b, bng, bnb, hw, hb):
    g = NPAD // NPT
    return pl.pallas_call(
        _post_body,
        grid=(g,),
        in_specs=[
            pl.BlockSpec((NPT, D), lambda i: (i, 0)),
            pl.BlockSpec((NPT, D), lambda i: (i, 0)),
            pl.BlockSpec((NPT, D), lambda i: (i, 0)),
            pl.BlockSpec((1, D), lambda i: (0, 0)),
            pl.BlockSpec((1, D), lambda i: (0, 0)),
            pl.BlockSpec((1, D), lambda i: (0, 0)),
            pl.BlockSpec((D, 1), lambda i: (0, 0)),
            pl.BlockSpec((1, 1), lambda i: (0, 0)),
        ],
        out_specs=pl.BlockSpec((NPT, 1), lambda i: (i, 0)),
        out_shape=jax.ShapeDtypeStruct((NPAD, 1), f32),
    )(p0, p1, res, b, bng, bnb, hw, hb)


_MESH = plsc.VectorSubcoreMesh(core_axis_name="c", subcore_axis_name="s")

_DN = lax.GatherDimensionNumbers(offset_dims=(), collapsed_slice_dims=(0,),
                                 start_index_map=(0,))


def _splat(v, h):
    return lax.gather(v, jnp.full((16, 1), h, i32), _DN, (1,),
                      mode=lax.GatherScatterMode.PROMISE_IN_BOUNDS)


def _bits_to_f32(xi):
    return lax.bitcast_convert_type(xi, f32)


def _sc_a_body(src_hbm, dst_hbm, alst_hbm, aldt_hbm, zden_hbm,
               den_out, e1d_out,
               sv0, sv1, dv0, dv1, as0, as1, ad0, ad1,
               eb0, eb1, ev0, ev1, den_sh,
               gs0, gs1, ss0, ss1, es0, es1):
    c = lax.axis_index("c")
    s = lax.axis_index("s")
    wid = c * 16 + s
    pltpu.sync_copy(zden_hbm, den_sh.at[pl.ds(s * NPT, NPT)])

    def zr(j, _):
        for k in range(1, D // 16):
            eb0[j, pl.ds(k * 16, 16)] = jnp.zeros((16,), f32)
            eb1[j, pl.ds(k * 16, 16)] = jnp.zeros((16,), f32)
        return 0

    lax.fori_loop(0, CA, zr, 0)
    plsc.subcore_barrier()
    base = wid * EPT

    def issue(off, sv, dvp, p, ag, ad, gs):
        pltpu.sync_copy(src_hbm.at[pl.ds(off, CA)], sv)
        pltpu.sync_copy(dst_hbm.at[pl.ds(off, CA)], dvp.at[p])
        pltpu.async_copy(alst_hbm.at[sv], ag, gs)
        pltpu.async_copy(aldt_hbm.at[dvp.at[p]], ad, gs)

    issue(base, sv0, dv0, 0, as0, ad0, gs0)
    issue(base + CA, sv1, dv1, 0, as1, ad1, gs1)

    def step(i, half, sv, dvp, ag, ad, eb, ev, gs, ss, es):
        ci = 2 * i + half
        p = jnp.bitwise_and(i, 1)
        off = base + ci * CA
        pltpu.make_async_copy(alst_hbm.at[sv], ag, gs).wait()
        pltpu.make_async_copy(aldt_hbm.at[dvp.at[p]], ad, gs).wait()

        @pl.when(i > 0)
        def _():
            pltpu.make_async_copy(eb, den_sh.at[dvp.at[1 - p]], ss).wait()
            pltpu.make_async_copy(
                ev, e1d_out.at[pl.ds(0, CA * 8)], es).wait()

        @plsc.parallel_loop(0, CA, unroll=4)
        def _(j):
            v = ag[j, pl.ds(0, 16)] + ad[j, pl.ds(0, 16)]
            v = jnp.where(v > 0, v, 0.2 * v)
            eb[j, pl.ds(0, 16)] = jnp.exp(v)

        @plsc.parallel_loop(0, CA // 2, unroll=2)
        def _(p):
            lane = lax.iota(i32, 16)
            a = eb[2 * p, pl.ds(0, 16)]
            bvec = eb[2 * p + 1, pl.ds(0, 16)]
            sh = lax.gather(bvec, jnp.maximum(lane - 8, 0).reshape(16, 1),
                            _DN, (1,),
                            mode=lax.GatherScatterMode.PROMISE_IN_BOUNDS)
            ev[pl.ds(p * 16, 16)] = jnp.where(lane < 8, a, sh)

        pltpu.async_copy(eb, den_sh.at[dvp.at[p]], ss, add=True)
        pltpu.async_copy(ev, e1d_out.at[pl.ds(off * 8, CA * 8)], es)

        @pl.when(ci + 2 < NA)
        def _():
            issue(off + 2 * CA, sv, dvp, 1 - p, ag, ad, gs)

    def it(i, _):
        step(i, 0, sv0, dv0, as0, ad0, eb0, ev0, gs0, ss0, es0)
        step(i, 1, sv1, dv1, as1, ad1, eb1, ev1, gs1, ss1, es1)
        return 0

    lax.fori_loop(0, NA // 2, it, 0)
    pltpu.make_async_copy(eb0, den_sh.at[dv0.at[0]], ss0).wait()
    pltpu.make_async_copy(eb1, den_sh.at[dv1.at[0]], ss1).wait()
    pltpu.make_async_copy(ev0, e1d_out.at[pl.ds(0, CA * 8)], es0).wait()
    pltpu.make_async_copy(ev1, e1d_out.at[pl.ds(0, CA * 8)], es1).wait()
    plsc.subcore_barrier()
    pltpu.sync_copy(den_sh.at[pl.ds(s * NPT, NPT)],
                    den_out.at[c, pl.ds(s * NPT, NPT)])


def _sc_a(src, dst, alst, aldt, zden):
    return pl.kernel(
        _sc_a_body,
        out_type=(
            jax.ShapeDtypeStruct((2, NPAD, D), f32),
            jax.ShapeDtypeStruct((EPAD * 8,), f32),
        ),
        mesh=_MESH,
        scratch_types=[
            pltpu.VMEM((CA,), i32), pltpu.VMEM((CA,), i32),
            pltpu.VMEM((2, CA), i32), pltpu.VMEM((2, CA), i32),
            pltpu.VMEM((CA, D), f32), pltpu.VMEM((CA, D), f32),
            pltpu.VMEM((CA, D), f32), pltpu.VMEM((CA, D), f32),
            pltpu.VMEM((CA, D), f32), pltpu.VMEM((CA, D), f32),
            pltpu.VMEM((CA * 8,), f32), pltpu.VMEM((CA * 8,), f32),
            pltpu.VMEM_SHARED((NPAD, D), f32),
            pltpu.SemaphoreType.DMA, pltpu.SemaphoreType.DMA,
            pltpu.SemaphoreType.DMA, pltpu.SemaphoreType.DMA,
            pltpu.SemaphoreType.DMA, pltpu.SemaphoreType.DMA,
        ],
    )(src, dst, alst, aldt, zden)


def _sc_b_body(src_hbm, dst_hbm, xpk_hbm, rden_hbm, e1d_hbm, zacc_hbm,
               part_out,
               sv0, sv1, dv0, dv1, gb0, gb1, ev0, ev1,
               ad0, ad1, mg0, mg1,
               acc_sh, gs0, gs1, ss0, ss1):
    c = lax.axis_index("c")
    s = lax.axis_index("s")
    wid = c * 16 + s
    pltpu.sync_copy(zacc_hbm, acc_sh.at[pl.ds(s * NPT, NPT)])
    plsc.subcore_barrier()
    base = wid * EPT

    def issue(off, sv, dvp, p, gb, evb, ad, gs):
        pltpu.sync_copy(src_hbm.at[pl.ds(off, CB)], sv)
        pltpu.sync_copy(dst_hbm.at[pl.ds(off, CB)], dvp.at[p])
        pltpu.async_copy(xpk_hbm.at[sv], gb, gs)
        pltpu.async_copy(e1d_hbm.at[pl.ds(off * 8, CB * 8)],
                         evb.at[pl.ds(0, CB * 8)], gs)
        pltpu.async_copy(rden_hbm.at[dvp.at[p]], ad, gs)

    issue(base, sv0, dv0, 0, gb0, ev0, ad0, gs0)
    issue(base + CB, sv1, dv1, 0, gb1, ev1, ad1, gs1)

    def step(i, half, sv, dvp, gb, evb, ad, mg, gs, ss):
        ci = 2 * i + half
        p = jnp.bitwise_and(i, 1)
        off = base + ci * CB
        pltpu.make_async_copy(xpk_hbm.at[sv], gb, gs).wait()
        pltpu.make_async_copy(e1d_hbm.at[pl.ds(off * 8, CB * 8)],
                              evb.at[pl.ds(0, CB * 8)], gs).wait()
        pltpu.make_async_copy(rden_hbm.at[dvp.at[p]], ad, gs).wait()

        @pl.when(i > 0)
        def _():
            pltpu.make_async_copy(mg, acc_sh.at[dvp.at[1 - p]], ss).wait()

        @plsc.parallel_loop(0, CB, unroll=2)
        def _(j):
            coef = evb[pl.ds(j * 8, 16)] * ad[j, pl.ds(0, 16)]
            chs = [_splat(coef, h) for h in range(H)]
            for w in range(4):
                acc_e = None
                acc_o = None
                for h in range(H):
                    flat = h * 64 + w * 16
                    xi = gb[j, flat // 128, pl.ds(flat % 128, 16)]
                    ev = _bits_to_f32(lax.shift_left(xi, 16))
                    od = _bits_to_f32(jnp.bitwise_and(xi, jnp.int32(-65536)))
                    if h == 0:
                        acc_e = chs[h] * ev
                        acc_o = chs[h] * od
                    else:
                        acc_e = acc_e + chs[h] * ev
                        acc_o = acc_o + chs[h] * od
                mg[j, pl.ds(w * 32, 16)] = acc_e
                mg[j, pl.ds(w * 32 + 16, 16)] = acc_o

        pltpu.async_copy(mg, acc_sh.at[dvp.at[p]], ss, add=True)

        @pl.when(ci + 2 < NB)
        def _():
            issue(off + 2 * CB, sv, dvp, 1 - p, gb, evb, ad, gs)

    def it(i, _):
        step(i, 0, sv0, dv0, gb0, ev0, ad0, mg0, gs0, ss0)
        step(i, 1, sv1, dv1, gb1, ev1, ad1, mg1, gs1, ss1)
        return 0

    lax.fori_loop(0, NB // 2, it, 0)
    pltpu.make_async_copy(mg0, acc_sh.at[dv0.at[0]], ss0).wait()
    pltpu.make_async_copy(mg1, acc_sh.at[dv1.at[0]], ss1).wait()
    plsc.subcore_barrier()
    pltpu.sync_copy(acc_sh.at[pl.ds(s * NPT, NPT)],
                    part_out.at[c, pl.ds(s * NPT, NPT)])


def _sc_b(src, dst, xpk, rden, e1d, zacc):
    return pl.kernel(
        _sc_b_body,
        out_type=jax.ShapeDtypeStruct((2, NPAD, D), f32),
        mesh=_MESH,
        scratch_types=[
            pltpu.VMEM((CB,), i32), pltpu.VMEM((CB,), i32),
            pltpu.VMEM((2, CB), i32), pltpu.VMEM((2, CB), i32),
            pltpu.VMEM((CB, PK // 128, 128), i32),
            pltpu.VMEM((CB, PK // 128, 128), i32),
            pltpu.VMEM((CB * 8 + 16,), f32), pltpu.VMEM((CB * 8 + 16,), f32),
            pltpu.VMEM((CB, D), f32), pltpu.VMEM((CB, D), f32),
            pltpu.VMEM((CB, D), f32), pltpu.VMEM((CB, D), f32),
            pltpu.VMEM_SHARED((NPAD, D), f32),
            pltpu.SemaphoreType.DMA, pltpu.SemaphoreType.DMA,
            pltpu.SemaphoreType.DMA, pltpu.SemaphoreType.DMA,
        ],
    )(src, dst, xpk, rden, e1d, zacc)


def _perm_cols(w):
    return w.reshape(-1, H, 4, 2, 16).transpose(0, 1, 2, 4, 3).reshape(w.shape)


def kernel(ctrl_expr, perturbation_mask, edge_index, ip_W, ip_b, ln_g, ln_b,
           W1, as1, ad1, b1, bn1_g, bn1_b,
           W2, as2, ad2, b2, bn2_g, bn2_b,
           W3, as3, ad3, b3, bn3_g, bn3_b,
           res_W, res_b, head_W, head_b):
    ctrl = jnp.zeros((NPAD, 1), f32).at[:N, 0].set(ctrl_expr)
    mask = jnp.zeros((NPAD, 1), f32).at[:N, 0].set(perturbation_mask)
    loop = jnp.arange(N, dtype=i32)
    padi = jnp.full((EPAD - EN,), DUM, i32)
    src = jnp.concatenate([edge_index[0].astype(i32), loop, padi])
    dst = jnp.concatenate([edge_index[1].astype(i32), loop, padi])
    zacc = jnp.zeros((NPT, D), f32)
    row = lambda v: v.reshape(1, -1)

    x0, resid = _tc_pre(ctrl, mask, ip_W, row(ip_b), row(ln_g), row(ln_b),
                        res_W, row(res_b))

    x = x0
    parts = None
    layers = [(W1, as1, ad1, b1, bn1_g, bn1_b),
              (W2, as2, ad2, b2, bn2_g, bn2_b),
              (W3, as3, ad3, b3, bn3_g, bn3_b)]
    for li, (W, a_s, a_d, b, bg, bb) in enumerate(layers):
        if li > 0:
            _, _, _, b_, bg_, bb_ = layers[li - 1]
            x = _tc_combine(parts[0], parts[1], row(b_), row(bg_), row(bb_))
        Wp = _perm_cols(W)
        asp = _perm_cols(a_s.reshape(1, -1))
        adp = _perm_cols(a_d.reshape(1, -1))
        xpb, alst, aldt = _tc_lpre(x, Wp, asp, adp)
        xpk = lax.bitcast_convert_type(
            xpb.reshape(NPAD, PK, 2), i32).reshape(NPAD, PK // 128, 128)
        den, e1d = _sc_a(src, dst, alst, aldt, zacc)
        rden = _tc_rden(den[0], den[1])
        parts = _sc_b(src, dst, xpk, rden, e1d, zacc)

    out = _tc_post(parts[0], parts[1], resid, row(b3), row(bn3_g), row(bn3_b),
                   head_W, row(head_b))
    return out[:N, 0]


# speedup vs baseline: 1.5701x; 1.2607x over previous
import math

import numpy as np
import jax
import jax.numpy as jnp
from jax import lax
from jax.experimental import pallas as pl
from jax.experimental.pallas import tpu as pltpu
from jax.experimental.pallas import tpu_sc as plsc

N = 10000
D = 128
H = 8
E = 320000
EN = E + N
NPAD = 10112
EPAD = 330240
EPT = EPAD // 32
NPT = NPAD // 16
CA = 40
NA = EPT // CA
CB = 24
NB = EPT // CB
PK = H * D // 2
DUM = N
_BN = 1.0 / math.sqrt(1.0 + 1e-5)

f32 = jnp.float32
i32 = jnp.int32
bf16 = jnp.bfloat16


def _elu(x):
    return jnp.where(x > 0, x, jnp.exp(jnp.minimum(x, 0.0)) - 1.0)


def _pre_body(ctrl, mask, ipw, ipb, lng, lnb, rw, rb, x0_o, res_o):
    t = ctrl[:, :] * ipw[:, :] + ipb[:, :]
    m = jnp.mean(t, axis=-1, keepdims=True)
    v = jnp.mean((t - m) ** 2, axis=-1, keepdims=True)
    t = (t - m) / jnp.sqrt(v + 1e-5) * lng[:, :] + lnb[:, :]
    x0 = _elu(t) * mask[:, :]
    x0_o[:, :] = x0
    res_o[:, :] = jnp.dot(x0, rw[:, :], preferred_element_type=f32) + rb[:, :]


def _tc_pre(ctrl, mask, ipw, ipb, lng, lnb, rw, rb):
    g = NPAD // NPT
    return pl.pallas_call(
        _pre_body,
        grid=(g,),
        in_specs=[
            pl.BlockSpec((NPT, 1), lambda i: (i, 0)),
            pl.BlockSpec((NPT, 1), lambda i: (i, 0)),
            pl.BlockSpec((1, D), lambda i: (0, 0)),
            pl.BlockSpec((1, D), lambda i: (0, 0)),
            pl.BlockSpec((1, D), lambda i: (0, 0)),
            pl.BlockSpec((1, D), lambda i: (0, 0)),
            pl.BlockSpec((D, D), lambda i: (0, 0)),
            pl.BlockSpec((1, D), lambda i: (0, 0)),
        ],
        out_specs=[
            pl.BlockSpec((NPT, D), lambda i: (i, 0)),
            pl.BlockSpec((NPT, D), lambda i: (i, 0)),
        ],
        out_shape=[
            jax.ShapeDtypeStruct((NPAD, D), f32),
            jax.ShapeDtypeStruct((NPAD, D), f32),
        ],
    )(ctrl, mask, ipw, ipb, lng, lnb, rw, rb)


def _lpre_body(x, w, asf, adf, xpb_o, als_o, ald_o):
    xp = jnp.dot(x[:, :], w[:, :], preferred_element_type=f32)
    xpb_o[:, :] = xp.astype(bf16)
    r = lax.broadcasted_iota(i32, (H * D, D), 0) // D
    c = lax.broadcasted_iota(i32, (H * D, D), 1)
    sel = (r == c).astype(f32)
    als_o[:, :] = jnp.dot(xp * asf[:, :], sel, preferred_element_type=f32)
    ald_o[:, :] = jnp.dot(xp * adf[:, :], sel, preferred_element_type=f32)


def _tc_lpre(x, w, asf, adf):
    g = NPAD // NPT
    return pl.pallas_call(
        _lpre_body,
        grid=(g,),
        in_specs=[
            pl.BlockSpec((NPT, D), lambda i: (i, 0)),
            pl.BlockSpec((D, H * D), lambda i: (0, 0)),
            pl.BlockSpec((1, H * D), lambda i: (0, 0)),
            pl.BlockSpec((1, H * D), lambda i: (0, 0)),
        ],
        out_specs=[
            pl.BlockSpec((NPT, H * D), lambda i: (i, 0)),
            pl.BlockSpec((NPT, D), lambda i: (i, 0)),
            pl.BlockSpec((NPT, D), lambda i: (i, 0)),
        ],
        out_shape=[
            jax.ShapeDtypeStruct((NPAD, H * D), bf16),
            jax.ShapeDtypeStruct((NPAD, D), f32),
            jax.ShapeDtypeStruct((NPAD, D), f32),
        ],
    )(x, w, asf, adf)


def _rden_body(d0, d1, o):
    o[:, :] = 1.0 / (d0[:, :] + d1[:, :] + 1e-16)


def _tc_rden(d0, d1):
    g = NPAD // NPT
    return pl.pallas_call(
        _rden_body,
        grid=(g,),
        in_specs=[
            pl.BlockSpec((NPT, D), lambda i: (i, 0)),
            pl.BlockSpec((NPT, D), lambda i: (i, 0)),
        ],
        out_specs=pl.BlockSpec((NPT, D), lambda i: (i, 0)),
        out_shape=jax.ShapeDtypeStruct((NPAD, D), f32),
    )(d0, d1)


def _comb_body(p0, p1, b, g_, b_, o):
    agg = (p0[:, :] + p1[:, :]) * (1.0 / H) + b[:, :]
    o[:, :] = _elu(agg * (_BN * g_[:, :]) + b_[:, :])


def _tc_combine(p0, p1, b, bng, bnb):
    g = NPAD // NPT
    return pl.pallas_call(
        _comb_body,
        grid=(g,),
        in_specs=[
            pl.BlockSpec((NPT, D), lambda i: (i, 0)),
            pl.BlockSpec((NPT, D), lambda i: (i, 0)),
            pl.BlockSpec((1, D), lambda i: (0, 0)),
            pl.BlockSpec((1, D), lambda i: (0, 0)),
            pl.BlockSpec((1, D), lambda i: (0, 0)),
        ],
        out_specs=pl.BlockSpec((NPT, D), lambda i: (i, 0)),
        out_shape=jax.ShapeDtypeStruct((NPAD, D), f32),
    )(p0, p1, b, bng, bnb)


def _post_body(p0, p1, res, b, g_, b_, hw, hb, o):
    agg = (p0[:, :] + p1[:, :]) * (1.0 / H) + b[:, :]
    x3 = agg * (_BN * g_[:, :]) + b_[:, :]
    z = _elu(x3 + res[:, :])
    o[:, :] = jnp.dot(z, hw[:, :], preferred_element_type=f32) + hb[:, :]


def _tc_post(p0, p1, res, b, bng, bnb, hw, hb):
    g = NPAD // NPT
    return pl.pallas_call(
        _post_body,
        grid=(g,),
        in_specs=[
            pl.BlockSpec((NPT, D), lambda i: (i, 0)),
            pl.BlockSpec((NPT, D), lambda i: (i, 0)),
            pl.BlockSpec((NPT, D), lambda i: (i, 0)),
            pl.BlockSpec((1, D), lambda i: (0, 0)),
            pl.BlockSpec((1, D), lambda i: (0, 0)),
            pl.BlockSpec((1, D), lambda i: (0, 0)),
            pl.BlockSpec((D, 1), lambda i: (0, 0)),
            pl.BlockSpec((1, 1), lambda i: (0, 0)),
        ],
        out_specs=pl.BlockSpec((NPT, 1), lambda i: (i, 0)),
        out_shape=jax.ShapeDtypeStruct((NPAD, 1), f32),
    )(p0, p1, res, b, bng, bnb, hw, hb)


_MESH = plsc.VectorSubcoreMesh(core_axis_name="c", subcore_axis_name="s")

_DN = lax.GatherDimensionNumbers(offset_dims=(), collapsed_slice_dims=(0,),
                                 start_index_map=(0,))


def _splat(v, h):
    return lax.gather(v, jnp.full((16, 1), h, i32), _DN, (1,),
                      mode=lax.GatherScatterMode.PROMISE_IN_BOUNDS)


def _bits_to_f32(xi):
    return lax.bitcast_convert_type(xi, f32)


def _sc_a_body(src_hbm, dst_hbm, alst_hbm, aldt_hbm, zden_hbm,
               den_out, e1d_out,
               sv0, sv1, dv0, dv1, as0, as1, ad0, ad1,
               eb0, eb1, ev0, ev1, den_sh,
               gs0, gs1, ss0, ss1, es0, es1, ia0, ia1):
    c = lax.axis_index("c")
    s = lax.axis_index("s")
    wid = c * 16 + s
    pltpu.sync_copy(zden_hbm, den_sh.at[pl.ds(s * NPT, NPT)])

    def zr(j, _):
        for k in range(1, D // 16):
            eb0[j, pl.ds(k * 16, 16)] = jnp.zeros((16,), f32)
            eb1[j, pl.ds(k * 16, 16)] = jnp.zeros((16,), f32)
        return 0

    lax.fori_loop(0, CA, zr, 0)
    plsc.subcore_barrier()
    base = wid * EPT

    def gathers(sv, dvp, p, ag, ad, gs):
        pltpu.async_copy(alst_hbm.at[sv], ag, gs)
        pltpu.async_copy(aldt_hbm.at[dvp.at[p]], ad, gs)

    def issue(off, sv, dvp, p, ag, ad, gs):
        pltpu.sync_copy(src_hbm.at[pl.ds(off, CA)], sv)
        pltpu.sync_copy(dst_hbm.at[pl.ds(off, CA)], dvp.at[p])
        gathers(sv, dvp, p, ag, ad, gs)

    issue(base, sv0, dv0, 0, as0, ad0, gs0)
    issue(base + CA, sv1, dv1, 0, as1, ad1, gs1)

    def step(i, half, sv, dvp, ag, ad, eb, ev, gs, ss, es, isem):
        ci = 2 * i + half
        p = jnp.bitwise_and(i, 1)
        off = base + ci * CA
        pltpu.make_async_copy(alst_hbm.at[sv], ag, gs).wait()
        pltpu.make_async_copy(aldt_hbm.at[dvp.at[p]], ad, gs).wait()

        @pl.when(i > 0)
        def _():
            pltpu.make_async_copy(eb, den_sh.at[dvp.at[1 - p]], ss).wait()
            pltpu.make_async_copy(
                ev, e1d_out.at[pl.ds(0, CA * 8)], es).wait()

        @pl.when(ci + 2 < NA)
        def _():
            pltpu.async_copy(src_hbm.at[pl.ds(off + 2 * CA, CA)], sv, isem)
            pltpu.async_copy(dst_hbm.at[pl.ds(off + 2 * CA, CA)],
                             dvp.at[1 - p], isem)

        @plsc.parallel_loop(0, CA, unroll=4)
        def _(j):
            v = ag[j, pl.ds(0, 16)] + ad[j, pl.ds(0, 16)]
            v = jnp.where(v > 0, v, 0.2 * v)
            eb[j, pl.ds(0, 16)] = jnp.exp(v)

        @plsc.parallel_loop(0, CA // 2, unroll=2)
        def _(p):
            lane = lax.iota(i32, 16)
            a = eb[2 * p, pl.ds(0, 16)]
            bvec = eb[2 * p + 1, pl.ds(0, 16)]
            sh = lax.gather(bvec, jnp.maximum(lane - 8, 0).reshape(16, 1),
                            _DN, (1,),
                            mode=lax.GatherScatterMode.PROMISE_IN_BOUNDS)
            ev[pl.ds(p * 16, 16)] = jnp.where(lane < 8, a, sh)

        pltpu.async_copy(eb, den_sh.at[dvp.at[p]], ss, add=True)
        pltpu.async_copy(ev, e1d_out.at[pl.ds(off * 8, CA * 8)], es)

        @pl.when(ci + 2 < NA)
        def _():
            pltpu.make_async_copy(src_hbm.at[pl.ds(off + 2 * CA, CA)],
                                  sv, isem).wait()
            pltpu.make_async_copy(dst_hbm.at[pl.ds(off + 2 * CA, CA)],
                                  dvp.at[1 - p], isem).wait()
            gathers(sv, dvp, 1 - p, ag, ad, gs)

    def it(i, _):
        step(i, 0, sv0, dv0, as0, ad0, eb0, ev0, gs0, ss0, es0, ia0)
        step(i, 1, sv1, dv1, as1, ad1, eb1, ev1, gs1, ss1, es1, ia1)
        return 0

    lax.fori_loop(0, NA // 2, it, 0)
    pltpu.make_async_copy(eb0, den_sh.at[dv0.at[0]], ss0).wait()
    pltpu.make_async_copy(eb1, den_sh.at[dv1.at[0]], ss1).wait()
    pltpu.make_async_copy(ev0, e1d_out.at[pl.ds(0, CA * 8)], es0).wait()
    pltpu.make_async_copy(ev1, e1d_out.at[pl.ds(0, CA * 8)], es1).wait()
    plsc.subcore_barrier()
    pltpu.sync_copy(den_sh.at[pl.ds(s * NPT, NPT)],
                    den_out.at[c, pl.ds(s * NPT, NPT)])


def _sc_a(src, dst, alst, aldt, zden):
    return pl.kernel(
        _sc_a_body,
        out_type=(
            jax.ShapeDtypeStruct((2, NPAD, D), f32),
            jax.ShapeDtypeStruct((EPAD * 8,), f32),
        ),
        mesh=_MESH,
        scratch_types=[
            pltpu.VMEM((CA,), i32), pltpu.VMEM((CA,), i32),
            pltpu.VMEM((2, CA), i32), pltpu.VMEM((2, CA), i32),
            pltpu.VMEM((CA, D), f32), pltpu.VMEM((CA, D), f32),
            pltpu.VMEM((CA, D), f32), pltpu.VMEM((CA, D), f32),
            pltpu.VMEM((CA, D), f32), pltpu.VMEM((CA, D), f32),
            pltpu.VMEM((CA * 8,), f32), pltpu.VMEM((CA * 8,), f32),
            pltpu.VMEM_SHARED((NPAD, D), f32),
            pltpu.SemaphoreType.DMA, pltpu.SemaphoreType.DMA,
            pltpu.SemaphoreType.DMA, pltpu.SemaphoreType.DMA,
            pltpu.SemaphoreType.DMA, pltpu.SemaphoreType.DMA,
            pltpu.SemaphoreType.DMA, pltpu.SemaphoreType.DMA,
        ],
    )(src, dst, alst, aldt, zden)


def _sc_b_body(src_hbm, dst_hbm, xpk_hbm, rden_hbm, e1d_hbm, zacc_hbm,
               part_out,
               sv0, sv1, dv0, dv1, gb0, gb1, ev0, ev1,
               ad0, ad1, mg0, mg1,
               acc_sh, gs0, gs1, ss0, ss1, is0, is1):
    c = lax.axis_index("c")
    s = lax.axis_index("s")
    wid = c * 16 + s
    pltpu.sync_copy(zacc_hbm, acc_sh.at[pl.ds(s * NPT, NPT)])
    plsc.subcore_barrier()
    base = wid * EPT

    def gathers(off, sv, dvp, p, gb, evb, ad, gs):
        pltpu.async_copy(xpk_hbm.at[sv], gb, gs)
        pltpu.async_copy(e1d_hbm.at[pl.ds(off * 8, CB * 8)],
                         evb.at[pl.ds(0, CB * 8)], gs)
        pltpu.async_copy(rden_hbm.at[dvp.at[p]], ad, gs)

    def issue(off, sv, dvp, p, gb, evb, ad, gs):
        pltpu.sync_copy(src_hbm.at[pl.ds(off, CB)], sv)
        pltpu.sync_copy(dst_hbm.at[pl.ds(off, CB)], dvp.at[p])
        gathers(off, sv, dvp, p, gb, evb, ad, gs)

    issue(base, sv0, dv0, 0, gb0, ev0, ad0, gs0)
    issue(base + CB, sv1, dv1, 0, gb1, ev1, ad1, gs1)

    def step(i, half, sv, dvp, gb, evb, ad, mg, gs, ss, isem):
        ci = 2 * i + half
        p = jnp.bitwise_and(i, 1)
        off = base + ci * CB
        pltpu.make_async_copy(xpk_hbm.at[sv], gb, gs).wait()
        pltpu.make_async_copy(e1d_hbm.at[pl.ds(off * 8, CB * 8)],
                              evb.at[pl.ds(0, CB * 8)], gs).wait()
        pltpu.make_async_copy(rden_hbm.at[dvp.at[p]], ad, gs).wait()

        @pl.when(i > 0)
        def _():
            pltpu.make_async_copy(mg, acc_sh.at[dvp.at[1 - p]], ss).wait()

        @pl.when(ci + 2 < NB)
        def _():
            pltpu.async_copy(src_hbm.at[pl.ds(off + 2 * CB, CB)], sv, isem)
            pltpu.async_copy(dst_hbm.at[pl.ds(off + 2 * CB, CB)],
                             dvp.at[1 - p], isem)

        @plsc.parallel_loop(0, CB, unroll=2)
        def _(j):
            coef = evb[pl.ds(j * 8, 16)] * ad[j, pl.ds(0, 16)]
            chs = [_splat(coef, h) for h in range(H)]
            for w in range(4):
                acc_e = None
                acc_o = None
                for h in range(H):
                    flat = h * 64 + w * 16
                    xi = gb[j, flat // 128, pl.ds(flat % 128, 16)]
                    ev = _bits_to_f32(lax.shift_left(xi, 16))
                    od = _bits_to_f32(jnp.bitwise_and(xi, jnp.int32(-65536)))
                    if h == 0:
                        acc_e = chs[h] * ev
                        acc_o = chs[h] * od
                    else:
                        acc_e = acc_e + chs[h] * ev
                        acc_o = acc_o + chs[h] * od
                mg[j, pl.ds(w * 32, 16)] = acc_e
                mg[j, pl.ds(w * 32 + 16, 16)] = acc_o

        pltpu.async_copy(mg, acc_sh.at[dvp.at[p]], ss, add=True)

        @pl.when(ci + 2 < NB)
        def _():
            pltpu.make_async_copy(src_hbm.at[pl.ds(off + 2 * CB, CB)],
                                  sv, isem).wait()
            pltpu.make_async_copy(dst_hbm.at[pl.ds(off + 2 * CB, CB)],
                                  dvp.at[1 - p], isem).wait()
            gathers(off + 2 * CB, sv, dvp, 1 - p, gb, evb, ad, gs)

    def it(i, _):
        step(i, 0, sv0, dv0, gb0, ev0, ad0, mg0, gs0, ss0, is0)
        step(i, 1, sv1, dv1, gb1, ev1, ad1, mg1, gs1, ss1, is1)
        return 0

    lax.fori_loop(0, NB // 2, it, 0)
    pltpu.make_async_copy(mg0, acc_sh.at[dv0.at[0]], ss0).wait()
    pltpu.make_async_copy(mg1, acc_sh.at[dv1.at[0]], ss1).wait()
    plsc.subcore_barrier()
    pltpu.sync_copy(acc_sh.at[pl.ds(s * NPT, NPT)],
                    part_out.at[c, pl.ds(s * NPT, NPT)])


def _sc_b(src, dst, xpk, rden, e1d, zacc):
    return pl.kernel(
        _sc_b_body,
        out_type=jax.ShapeDtypeStruct((2, NPAD, D), f32),
        mesh=_MESH,
        scratch_types=[
            pltpu.VMEM((CB,), i32), pltpu.VMEM((CB,), i32),
            pltpu.VMEM((2, CB), i32), pltpu.VMEM((2, CB), i32),
            pltpu.VMEM((CB, PK // 128, 128), i32),
            pltpu.VMEM((CB, PK // 128, 128), i32),
            pltpu.VMEM((CB * 8 + 16,), f32), pltpu.VMEM((CB * 8 + 16,), f32),
            pltpu.VMEM((CB, D), f32), pltpu.VMEM((CB, D), f32),
            pltpu.VMEM((CB, D), f32), pltpu.VMEM((CB, D), f32),
            pltpu.VMEM_SHARED((NPAD, D), f32),
            pltpu.SemaphoreType.DMA, pltpu.SemaphoreType.DMA,
            pltpu.SemaphoreType.DMA, pltpu.SemaphoreType.DMA,
            pltpu.SemaphoreType.DMA, pltpu.SemaphoreType.DMA,
        ],
    )(src, dst, xpk, rden, e1d, zacc)


def _perm_cols(w):
    return w.reshape(-1, H, 4, 2, 16).transpose(0, 1, 2, 4, 3).reshape(w.shape)


def kernel(ctrl_expr, perturbation_mask, edge_index, ip_W, ip_b, ln_g, ln_b,
           W1, as1, ad1, b1, bn1_g, bn1_b,
           W2, as2, ad2, b2, bn2_g, bn2_b,
           W3, as3, ad3, b3, bn3_g, bn3_b,
           res_W, res_b, head_W, head_b):
    ctrl = jnp.zeros((NPAD, 1), f32).at[:N, 0].set(ctrl_expr)
    mask = jnp.zeros((NPAD, 1), f32).at[:N, 0].set(perturbation_mask)
    loop = jnp.arange(N, dtype=i32)
    padi = jnp.full((EPAD - EN,), DUM, i32)
    src = jnp.concatenate([edge_index[0].astype(i32), loop, padi])
    dst = jnp.concatenate([edge_index[1].astype(i32), loop, padi])
    zacc = jnp.zeros((NPT, D), f32)
    row = lambda v: v.reshape(1, -1)

    x0, resid = _tc_pre(ctrl, mask, ip_W, row(ip_b), row(ln_g), row(ln_b),
                        res_W, row(res_b))

    x = x0
    parts = None
    layers = [(W1, as1, ad1, b1, bn1_g, bn1_b),
              (W2, as2, ad2, b2, bn2_g, bn2_b),
              (W3, as3, ad3, b3, bn3_g, bn3_b)]
    for li, (W, a_s, a_d, b, bg, bb) in enumerate(layers):
        if li > 0:
            _, _, _, b_, bg_, bb_ = layers[li - 1]
            x = _tc_combine(parts[0], parts[1], row(b_), row(bg_), row(bb_))
        Wp = _perm_cols(W)
        asp = _perm_cols(a_s.reshape(1, -1))
        adp = _perm_cols(a_d.reshape(1, -1))
        xpb, alst, aldt = _tc_lpre(x, Wp, asp, adp)
        xpk = lax.bitcast_convert_type(
            xpb.reshape(NPAD, PK, 2), i32).reshape(NPAD, PK // 128, 128)
        den, e1d = _sc_a(src, dst, alst, aldt, zacc)
        rden = _tc_rden(den[0], den[1])
        parts = _sc_b(src, dst, xpk, rden, e1d, zacc)

    out = _tc_post(parts[0], parts[1], resid, row(b3), row(bn3_g), row(bn3_b),
                   head_W, row(head_b))
    return out[:N, 0]


# speedup vs baseline: 1.8159x; 1.1566x over previous
import math

import numpy as np
import jax
import jax.numpy as jnp
from jax import lax
from jax.experimental import pallas as pl
from jax.experimental.pallas import tpu as pltpu
from jax.experimental.pallas import tpu_sc as plsc

N = 10000
D = 128
H = 8
E = 320000
EN = E + N
NPAD = 10112
EPAD = 330240
EPT = EPAD // 32
NPT = NPAD // 16
CA = 40
NA = EPT // CA
CB = 24
NB = EPT // CB
PK = H * D // 2
DUM = N
_BN = 1.0 / math.sqrt(1.0 + 1e-5)

f32 = jnp.float32
i32 = jnp.int32
bf16 = jnp.bfloat16


def _elu(x):
    return jnp.where(x > 0, x, jnp.exp(jnp.minimum(x, 0.0)) - 1.0)


def _pre_body(ctrl, mask, ipw, ipb, lng, lnb, rw, rb, x0_o, res_o):
    t = ctrl[:, :] * ipw[:, :] + ipb[:, :]
    m = jnp.mean(t, axis=-1, keepdims=True)
    v = jnp.mean((t - m) ** 2, axis=-1, keepdims=True)
    t = (t - m) / jnp.sqrt(v + 1e-5) * lng[:, :] + lnb[:, :]
    x0 = _elu(t) * mask[:, :]
    x0_o[:, :] = x0
    res_o[:, :] = jnp.dot(x0, rw[:, :], preferred_element_type=f32) + rb[:, :]


def _tc_pre(ctrl, mask, ipw, ipb, lng, lnb, rw, rb):
    g = NPAD // NPT
    return pl.pallas_call(
        _pre_body,
        grid=(g,),
        in_specs=[
            pl.BlockSpec((NPT, 1), lambda i: (i, 0)),
            pl.BlockSpec((NPT, 1), lambda i: (i, 0)),
            pl.BlockSpec((1, D), lambda i: (0, 0)),
            pl.BlockSpec((1, D), lambda i: (0, 0)),
            pl.BlockSpec((1, D), lambda i: (0, 0)),
            pl.BlockSpec((1, D), lambda i: (0, 0)),
            pl.BlockSpec((D, D), lambda i: (0, 0)),
            pl.BlockSpec((1, D), lambda i: (0, 0)),
        ],
        out_specs=[
            pl.BlockSpec((NPT, D), lambda i: (i, 0)),
            pl.BlockSpec((NPT, D), lambda i: (i, 0)),
        ],
        out_shape=[
            jax.ShapeDtypeStruct((NPAD, D), f32),
            jax.ShapeDtypeStruct((NPAD, D), f32),
        ],
    )(ctrl, mask, ipw, ipb, lng, lnb, rw, rb)


def _lpre_tail(xp, asf, adf, xpk_o, als_o, ald_o):
    u = lax.bitcast_convert_type(xp, i32)
    r = u + 0x7FFF + jnp.bitwise_and(jnp.right_shift(u, 16), 1)
    lo = lax.shift_right_logical(r[:, :PK], 16)
    hi = jnp.bitwise_and(r[:, PK:], jnp.int32(-65536))
    xpk_o[:, :] = jnp.bitwise_or(lo, hi)
    hd = jnp.mod(lax.broadcasted_iota(i32, (H * D, D), 0) // 64, H)
    c = lax.broadcasted_iota(i32, (H * D, D), 1)
    sel = (hd == c).astype(f32)
    als_o[:, :] = jnp.dot(xp * asf[:, :], sel, preferred_element_type=f32)
    ald_o[:, :] = jnp.dot(xp * adf[:, :], sel, preferred_element_type=f32)


def _lpre_body(x, w, asf, adf, xpk_o, als_o, ald_o):
    xp = jnp.dot(x[:, :], w[:, :], preferred_element_type=f32)
    _lpre_tail(xp, asf, adf, xpk_o, als_o, ald_o)


def _clpre_body(p0, p1, b, g_, b_, w, asf, adf, xpk_o, als_o, ald_o):
    agg = (p0[:, :] + p1[:, :]) * (1.0 / H) + b[:, :]
    x = _elu(agg * (_BN * g_[:, :]) + b_[:, :])
    xp = jnp.dot(x, w[:, :], preferred_element_type=f32)
    _lpre_tail(xp, asf, adf, xpk_o, als_o, ald_o)


_LPRE_OUT = dict(
    out_specs=[
        pl.BlockSpec((NPT, PK), lambda i: (i, 0)),
        pl.BlockSpec((NPT, D), lambda i: (i, 0)),
        pl.BlockSpec((NPT, D), lambda i: (i, 0)),
    ],
    out_shape=[
        jax.ShapeDtypeStruct((NPAD, PK), i32),
        jax.ShapeDtypeStruct((NPAD, D), f32),
        jax.ShapeDtypeStruct((NPAD, D), f32),
    ],
)


def _tc_lpre(x, w, asf, adf):
    g = NPAD // NPT
    return pl.pallas_call(
        _lpre_body,
        grid=(g,),
        in_specs=[
            pl.BlockSpec((NPT, D), lambda i: (i, 0)),
            pl.BlockSpec((D, H * D), lambda i: (0, 0)),
            pl.BlockSpec((1, H * D), lambda i: (0, 0)),
            pl.BlockSpec((1, H * D), lambda i: (0, 0)),
        ],
        **_LPRE_OUT,
    )(x, w, asf, adf)


def _tc_clpre(p0, p1, b, bng, bnb, w, asf, adf):
    g = NPAD // NPT
    return pl.pallas_call(
        _clpre_body,
        grid=(g,),
        in_specs=[
            pl.BlockSpec((NPT, D), lambda i: (i, 0)),
            pl.BlockSpec((NPT, D), lambda i: (i, 0)),
            pl.BlockSpec((1, D), lambda i: (0, 0)),
            pl.BlockSpec((1, D), lambda i: (0, 0)),
            pl.BlockSpec((1, D), lambda i: (0, 0)),
            pl.BlockSpec((D, H * D), lambda i: (0, 0)),
            pl.BlockSpec((1, H * D), lambda i: (0, 0)),
            pl.BlockSpec((1, H * D), lambda i: (0, 0)),
        ],
        **_LPRE_OUT,
    )(p0, p1, b, bng, bnb, w, asf, adf)


def _rden_body(d0, d1, o):
    o[:, :] = 1.0 / (d0[:, :] + d1[:, :] + 1e-16)


def _tc_rden(d0, d1):
    g = NPAD // NPT
    return pl.pallas_call(
        _rden_body,
        grid=(g,),
        in_specs=[
            pl.BlockSpec((NPT, D), lambda i: (i, 0)),
            pl.BlockSpec((NPT, D), lambda i: (i, 0)),
        ],
        out_specs=pl.BlockSpec((NPT, D), lambda i: (i, 0)),
        out_shape=jax.ShapeDtypeStruct((NPAD, D), f32),
    )(d0, d1)


def _comb_body(p0, p1, b, g_, b_, o):
    agg = (p0[:, :] + p1[:, :]) * (1.0 / H) + b[:, :]
    o[:, :] = _elu(agg * (_BN * g_[:, :]) + b_[:, :])


def _tc_combine(p0, p1, b, bng, bnb):
    g = NPAD // NPT
    return pl.pallas_call(
        _comb_body,
        grid=(g,),
        in_specs=[
            pl.BlockSpec((NPT, D), lambda i: (i, 0)),
            pl.BlockSpec((NPT, D), lambda i: (i, 0)),
            pl.BlockSpec((1, D), lambda i: (0, 0)),
            pl.BlockSpec((1, D), lambda i: (0, 0)),
            pl.BlockSpec((1, D), lambda i: (0, 0)),
        ],
        out_specs=pl.BlockSpec((NPT, D), lambda i: (i, 0)),
        out_shape=jax.ShapeDtypeStruct((NPAD, D), f32),
    )(p0, p1, b, bng, bnb)


def _post_body(p0, p1, res, b, g_, b_, hw, hb, o):
    agg = (p0[:, :] + p1[:, :]) * (1.0 / H) + b[:, :]
    x3 = agg * (_BN * g_[:, :]) + b_[:, :]
    z = _elu(x3 + res[:, :])
    o[:, :] = jnp.dot(z, hw[:, :], preferred_element_type=f32) + hb[:, :]


def _tc_post(p0, p1, res, b, bng, bnb, hw, hb):
    g = NPAD // NPT
    return pl.pallas_call(
        _post_body,
        grid=(g,),
        in_specs=[
            pl.BlockSpec((NPT, D), lambda i: (i, 0)),
            pl.BlockSpec((NPT, D), lambda i: (i, 0)),
            pl.BlockSpec((NPT, D), lambda i: (i, 0)),
            pl.BlockSpec((1, D), lambda i: (0, 0)),
            pl.BlockSpec((1, D), lambda i: (0, 0)),
            pl.BlockSpec((1, D), lambda i: (0, 0)),
            pl.BlockSpec((D, 1), lambda i: (0, 0)),
            pl.BlockSpec((1, 1), lambda i: (0, 0)),
        ],
        out_specs=pl.BlockSpec((NPT, 1), lambda i: (i, 0)),
        out_shape=jax.ShapeDtypeStruct((NPAD, 1), f32),
    )(p0, p1, res, b, bng, bnb, hw, hb)


_MESH = plsc.VectorSubcoreMesh(core_axis_name="c", subcore_axis_name="s")

_DN = lax.GatherDimensionNumbers(offset_dims=(), collapsed_slice_dims=(0,),
                                 start_index_map=(0,))


def _splat(v, h):
    return lax.gather(v, jnp.full((16, 1), h, i32), _DN, (1,),
                      mode=lax.GatherScatterMode.PROMISE_IN_BOUNDS)


def _bits_to_f32(xi):
    return lax.bitcast_convert_type(xi, f32)


def _sc_a_body(src_hbm, dst_hbm, alst_hbm, aldt_hbm, zden_hbm,
               den_out, e1d_out,
               sv0, sv1, dv0, dv1, as0, as1, ad0, ad1,
               eb0, eb1, ev0, ev1, den_sh,
               gs0, gs1, ss0, ss1, es0, es1, ia0, ia1):
    c = lax.axis_index("c")
    s = lax.axis_index("s")
    wid = c * 16 + s
    pltpu.sync_copy(zden_hbm, den_sh.at[pl.ds(s * NPT, NPT)])

    def zr(j, _):
        for k in range(1, D // 16):
            eb0[j, pl.ds(k * 16, 16)] = jnp.zeros((16,), f32)
            eb1[j, pl.ds(k * 16, 16)] = jnp.zeros((16,), f32)
        return 0

    lax.fori_loop(0, CA, zr, 0)
    plsc.subcore_barrier()
    base = wid * EPT

    def gathers(sv, dvp, p, ag, ad, gs):
        pltpu.async_copy(alst_hbm.at[sv], ag, gs)
        pltpu.async_copy(aldt_hbm.at[dvp.at[p]], ad, gs)

    def issue(off, sv, dvp, p, ag, ad, gs):
        pltpu.sync_copy(src_hbm.at[pl.ds(off, CA)], sv)
        pltpu.sync_copy(dst_hbm.at[pl.ds(off, CA)], dvp.at[p])
        gathers(sv, dvp, p, ag, ad, gs)

    issue(base, sv0, dv0, 0, as0, ad0, gs0)
    issue(base + CA, sv1, dv1, 0, as1, ad1, gs1)

    def step(i, half, sv, dvp, ag, ad, eb, ev, gs, ss, es, isem):
        ci = 2 * i + half
        p = jnp.bitwise_and(i, 1)
        off = base + ci * CA
        pltpu.make_async_copy(alst_hbm.at[sv], ag, gs).wait()
        pltpu.make_async_copy(aldt_hbm.at[dvp.at[p]], ad, gs).wait()

        @pl.when(i > 0)
        def _():
            pltpu.make_async_copy(eb, den_sh.at[dvp.at[1 - p]], ss).wait()
            pltpu.make_async_copy(
                ev, e1d_out.at[pl.ds(0, CA * 8)], es).wait()

        @pl.when(ci + 2 < NA)
        def _():
            pltpu.async_copy(src_hbm.at[pl.ds(off + 2 * CA, CA)], sv, isem)
            pltpu.async_copy(dst_hbm.at[pl.ds(off + 2 * CA, CA)],
                             dvp.at[1 - p], isem)

        @plsc.parallel_loop(0, CA, unroll=4)
        def _(j):
            v = ag[j, pl.ds(0, 16)] + ad[j, pl.ds(0, 16)]
            v = jnp.where(v > 0, v, 0.2 * v)
            eb[j, pl.ds(0, 16)] = jnp.exp(v)

        @plsc.parallel_loop(0, CA // 2, unroll=2)
        def _(p):
            lane = lax.iota(i32, 16)
            a = eb[2 * p, pl.ds(0, 16)]
            bvec = eb[2 * p + 1, pl.ds(0, 16)]
            sh = lax.gather(bvec, jnp.maximum(lane - 8, 0).reshape(16, 1),
                            _DN, (1,),
                            mode=lax.GatherScatterMode.PROMISE_IN_BOUNDS)
            ev[pl.ds(p * 16, 16)] = jnp.where(lane < 8, a, sh)

        pltpu.async_copy(eb, den_sh.at[dvp.at[p]], ss, add=True)
        pltpu.async_copy(ev, e1d_out.at[pl.ds(off * 8, CA * 8)], es)

        @pl.when(ci + 2 < NA)
        def _():
            pltpu.make_async_copy(src_hbm.at[pl.ds(off + 2 * CA, CA)],
                                  sv, isem).wait()
            pltpu.make_async_copy(dst_hbm.at[pl.ds(off + 2 * CA, CA)],
                                  dvp.at[1 - p], isem).wait()
            gathers(sv, dvp, 1 - p, ag, ad, gs)

    def it(i, _):
        step(i, 0, sv0, dv0, as0, ad0, eb0, ev0, gs0, ss0, es0, ia0)
        step(i, 1, sv1, dv1, as1, ad1, eb1, ev1, gs1, ss1, es1, ia1)
        return 0

    lax.fori_loop(0, NA // 2, it, 0)
    pltpu.make_async_copy(eb0, den_sh.at[dv0.at[0]], ss0).wait()
    pltpu.make_async_copy(eb1, den_sh.at[dv1.at[0]], ss1).wait()
    pltpu.make_async_copy(ev0, e1d_out.at[pl.ds(0, CA * 8)], es0).wait()
    pltpu.make_async_copy(ev1, e1d_out.at[pl.ds(0, CA * 8)], es1).wait()
    plsc.subcore_barrier()
    pltpu.sync_copy(den_sh.at[pl.ds(s * NPT, NPT)],
                    den_out.at[c, pl.ds(s * NPT, NPT)])


def _sc_a(src, dst, alst, aldt, zden):
    return pl.kernel(
        _sc_a_body,
        out_type=(
            jax.ShapeDtypeStruct((2, NPAD, D), f32),
            jax.ShapeDtypeStruct((EPAD * 8,), f32),
        ),
        mesh=_MESH,
        scratch_types=[
            pltpu.VMEM((CA,), i32), pltpu.VMEM((CA,), i32),
            pltpu.VMEM((2, CA), i32), pltpu.VMEM((2, CA), i32),
            pltpu.VMEM((CA, D), f32), pltpu.VMEM((CA, D), f32),
            pltpu.VMEM((CA, D), f32), pltpu.VMEM((CA, D), f32),
            pltpu.VMEM((CA, D), f32), pltpu.VMEM((CA, D), f32),
            pltpu.VMEM((CA * 8,), f32), pltpu.VMEM((CA * 8,), f32),
            pltpu.VMEM_SHARED((NPAD, D), f32),
            pltpu.SemaphoreType.DMA, pltpu.SemaphoreType.DMA,
            pltpu.SemaphoreType.DMA, pltpu.SemaphoreType.DMA,
            pltpu.SemaphoreType.DMA, pltpu.SemaphoreType.DMA,
            pltpu.SemaphoreType.DMA, pltpu.SemaphoreType.DMA,
        ],
    )(src, dst, alst, aldt, zden)


def _sc_b_body(src_hbm, dst_hbm, xpk_hbm, rden_hbm, e1d_hbm, zacc_hbm,
               part_out,
               sv0, sv1, dv0, dv1, gb0, gb1, ev0, ev1,
               ad0, ad1, mg0, mg1,
               acc_sh, gs0, gs1, ss0, ss1, is0, is1):
    c = lax.axis_index("c")
    s = lax.axis_index("s")
    wid = c * 16 + s
    pltpu.sync_copy(zacc_hbm, acc_sh.at[pl.ds(s * NPT, NPT)])
    plsc.subcore_barrier()
    base = wid * EPT

    def gathers(off, sv, dvp, p, gb, evb, ad, gs):
        pltpu.async_copy(xpk_hbm.at[sv], gb, gs)
        pltpu.async_copy(e1d_hbm.at[pl.ds(off * 8, CB * 8)],
                         evb.at[pl.ds(0, CB * 8)], gs)
        pltpu.async_copy(rden_hbm.at[dvp.at[p]], ad, gs)

    def issue(off, sv, dvp, p, gb, evb, ad, gs):
        pltpu.sync_copy(src_hbm.at[pl.ds(off, CB)], sv)
        pltpu.sync_copy(dst_hbm.at[pl.ds(off, CB)], dvp.at[p])
        gathers(off, sv, dvp, p, gb, evb, ad, gs)

    issue(base, sv0, dv0, 0, gb0, ev0, ad0, gs0)
    issue(base + CB, sv1, dv1, 0, gb1, ev1, ad1, gs1)

    def step(i, half, sv, dvp, gb, evb, ad, mg, gs, ss, isem):
        ci = 2 * i + half
        p = jnp.bitwise_and(i, 1)
        off = base + ci * CB
        pltpu.make_async_copy(xpk_hbm.at[sv], gb, gs).wait()
        pltpu.make_async_copy(e1d_hbm.at[pl.ds(off * 8, CB * 8)],
                              evb.at[pl.ds(0, CB * 8)], gs).wait()
        pltpu.make_async_copy(rden_hbm.at[dvp.at[p]], ad, gs).wait()

        @pl.when(i > 0)
        def _():
            pltpu.make_async_copy(mg, acc_sh.at[dvp.at[1 - p]], ss).wait()

        @pl.when(ci + 2 < NB)
        def _():
            pltpu.async_copy(src_hbm.at[pl.ds(off + 2 * CB, CB)], sv, isem)
            pltpu.async_copy(dst_hbm.at[pl.ds(off + 2 * CB, CB)],
                             dvp.at[1 - p], isem)

        @plsc.parallel_loop(0, CB, unroll=2)
        def _(j):
            coef = evb[pl.ds(j * 8, 16)] * ad[j, pl.ds(0, 16)]
            chs = [_splat(coef, h) for h in range(H)]
            for w in range(4):
                acc_e = None
                acc_o = None
                for h in range(H):
                    flat = h * 64 + w * 16
                    xi = gb[j, flat // 128, pl.ds(flat % 128, 16)]
                    ev = _bits_to_f32(lax.shift_left(xi, 16))
                    od = _bits_to_f32(jnp.bitwise_and(xi, jnp.int32(-65536)))
                    if h == 0:
                        acc_e = chs[h] * ev
                        acc_o = chs[h] * od
                    else:
                        acc_e = acc_e + chs[h] * ev
                        acc_o = acc_o + chs[h] * od
                mg[j, pl.ds(w * 32, 16)] = acc_e
                mg[j, pl.ds(w * 32 + 16, 16)] = acc_o

        pltpu.async_copy(mg, acc_sh.at[dvp.at[p]], ss, add=True)

        @pl.when(ci + 2 < NB)
        def _():
            pltpu.make_async_copy(src_hbm.at[pl.ds(off + 2 * CB, CB)],
                                  sv, isem).wait()
            pltpu.make_async_copy(dst_hbm.at[pl.ds(off + 2 * CB, CB)],
                                  dvp.at[1 - p], isem).wait()
            gathers(off + 2 * CB, sv, dvp, 1 - p, gb, evb, ad, gs)

    def it(i, _):
        step(i, 0, sv0, dv0, gb0, ev0, ad0, mg0, gs0, ss0, is0)
        step(i, 1, sv1, dv1, gb1, ev1, ad1, mg1, gs1, ss1, is1)
        return 0

    lax.fori_loop(0, NB // 2, it, 0)
    pltpu.make_async_copy(mg0, acc_sh.at[dv0.at[0]], ss0).wait()
    pltpu.make_async_copy(mg1, acc_sh.at[dv1.at[0]], ss1).wait()
    plsc.subcore_barrier()
    pltpu.sync_copy(acc_sh.at[pl.ds(s * NPT, NPT)],
                    part_out.at[c, pl.ds(s * NPT, NPT)])


def _sc_b(src, dst, xpk, rden, e1d, zacc):
    return pl.kernel(
        _sc_b_body,
        out_type=jax.ShapeDtypeStruct((2, NPAD, D), f32),
        mesh=_MESH,
        scratch_types=[
            pltpu.VMEM((CB,), i32), pltpu.VMEM((CB,), i32),
            pltpu.VMEM((2, CB), i32), pltpu.VMEM((2, CB), i32),
            pltpu.VMEM((CB, PK // 128, 128), i32),
            pltpu.VMEM((CB, PK // 128, 128), i32),
            pltpu.VMEM((CB * 8 + 16,), f32), pltpu.VMEM((CB * 8 + 16,), f32),
            pltpu.VMEM((CB, D), f32), pltpu.VMEM((CB, D), f32),
            pltpu.VMEM((CB, D), f32), pltpu.VMEM((CB, D), f32),
            pltpu.VMEM_SHARED((NPAD, D), f32),
            pltpu.SemaphoreType.DMA, pltpu.SemaphoreType.DMA,
            pltpu.SemaphoreType.DMA, pltpu.SemaphoreType.DMA,
            pltpu.SemaphoreType.DMA, pltpu.SemaphoreType.DMA,
        ],
    )(src, dst, xpk, rden, e1d, zacc)


def _pack_perm():
    p = np.arange(H * D)
    q = np.where(p < PK, p, p - PK)
    h, w, t = q // 64, (q % 64) // 16, q % 16
    return h * 128 + 32 * w + t + np.where(p < PK, 0, 16)


_PERM = _pack_perm()


def _perm_cols(w):
    return w[:, _PERM] if w.ndim == 2 else w[_PERM]


def kernel(ctrl_expr, perturbation_mask, edge_index, ip_W, ip_b, ln_g, ln_b,
           W1, as1, ad1, b1, bn1_g, bn1_b,
           W2, as2, ad2, b2, bn2_g, bn2_b,
           W3, as3, ad3, b3, bn3_g, bn3_b,
           res_W, res_b, head_W, head_b):
    ctrl = jnp.zeros((NPAD, 1), f32).at[:N, 0].set(ctrl_expr)
    mask = jnp.zeros((NPAD, 1), f32).at[:N, 0].set(perturbation_mask)
    loop = jnp.arange(N, dtype=i32)
    padi = jnp.full((EPAD - EN,), DUM, i32)
    src = jnp.concatenate([edge_index[0].astype(i32), loop, padi])
    dst = jnp.concatenate([edge_index[1].astype(i32), loop, padi])
    zacc = jnp.zeros((NPT, D), f32)
    row = lambda v: v.reshape(1, -1)

    x0, resid = _tc_pre(ctrl, mask, ip_W, row(ip_b), row(ln_g), row(ln_b),
                        res_W, row(res_b))

    x = x0
    parts = None
    layers = [(W1, as1, ad1, b1, bn1_g, bn1_b),
              (W2, as2, ad2, b2, bn2_g, bn2_b),
              (W3, as3, ad3, b3, bn3_g, bn3_b)]
    for li, (W, a_s, a_d, b, bg, bb) in enumerate(layers):
        Wp = _perm_cols(W)
        asp = _perm_cols(a_s.reshape(-1)).reshape(1, -1)
        adp = _perm_cols(a_d.reshape(-1)).reshape(1, -1)
        if li > 0:
            _, _, _, b_, bg_, bb_ = layers[li - 1]
            xpk2, alst, aldt = _tc_clpre(parts[0], parts[1], row(b_),
                                         row(bg_), row(bb_), Wp, asp, adp)
        else:
            xpk2, alst, aldt = _tc_lpre(x, Wp, asp, adp)
        xpk = xpk2.reshape(NPAD, PK // 128, 128)
        den, e1d = _sc_a(src, dst, alst, aldt, zacc)
        rden = _tc_rden(den[0], den[1])
        parts = _sc_b(src, dst, xpk, rden, e1d, zacc)

    out = _tc_post(parts[0], parts[1], resid, row(b3), row(bn3_g), row(bn3_b),
                   head_W, row(head_b))
    return out[:N, 0]
